# Initial kernel scaffold; baseline (speedup 1.0000x reference)
import jax
import jax.numpy as jnp
from jax.experimental import pallas as pl


def kernel(x, edge_index, W1, b1, W2, b2):
    raise NotImplementedError("write your pallas kernel here")


# speedup vs baseline: 17.2709x; 17.2709x over previous
import functools

import jax
import jax.numpy as jnp
from jax import lax
from jax.experimental import pallas as pl
from jax.experimental.pallas import tpu as pltpu
import jax.experimental.pallas.tpu_sc as plsc

N_NODES = 10000
N_EDGES = 320000
D_IN = 128
NHID = 128
NCLASS = 40
D2PAD = 64

NW = 32
EPW = N_EDGES // NW
CH = 80
NCHUNK = EPW // CH
ZR = 200
NZCH = N_NODES // ZR


def _sc_mesh():
    return plsc.VectorSubcoreMesh(core_axis_name="c", subcore_axis_name="s")


def _for_my_chunks(s, fn):
    def body(i, carry):
        k = s + 16 * i

        @pl.when(k < NZCH)
        def _():
            fn(k * ZR)

        return carry

    lax.fori_loop(0, (NZCH + 15) // 16, body, 0)


def _zero_fill(buf, rows, cols):
    nvec = rows * (cols // 16)

    def body(i, carry):
        r = i // (cols // 16)
        k = i % (cols // 16)
        buf[r, pl.ds(k * 16, 16)] = jnp.zeros((16,), jnp.float32)
        return carry

    lax.fori_loop(0, nvec, body, 0)


def _make_deg_kernel():
    @functools.partial(
        pl.kernel,
        out_type=jax.ShapeDtypeStruct((2, N_NODES, 16), jnp.float32),
        mesh=_sc_mesh(),
        scratch_types=[
            pltpu.VMEM((NCHUNK, CH), jnp.int32),
            pltpu.VMEM((CH, 16), jnp.float32),
            pltpu.VMEM((ZR, 16), jnp.float32),
            pltpu.VMEM_SHARED((N_NODES, 16), jnp.float32),
        ],
        compiler_params=pltpu.CompilerParams(use_tc_tiling_on_sc=False),
    )
    def deg_kernel(dst_hbm, out_hbm, idx_v, ones_v, stage_v, acc_sh):
        c = lax.axis_index("c")
        s = lax.axis_index("s")
        wid = c * 16 + s

        _zero_fill(stage_v, ZR, 16)
        _for_my_chunks(s, lambda r0: pltpu.sync_copy(
            stage_v, acc_sh.at[pl.ds(r0, ZR)]))

        def ones_fill(i, carry):
            ones_v[i, :] = jnp.ones((16,), jnp.float32)
            return carry

        lax.fori_loop(0, CH, ones_fill, 0)

        pltpu.sync_copy(dst_hbm.at[wid], idx_v)
        plsc.subcore_barrier()

        def body(j, carry):
            pltpu.sync_copy(ones_v, acc_sh.at[idx_v.at[j]], add=True)
            return carry

        lax.fori_loop(0, NCHUNK, body, 0)
        plsc.subcore_barrier()

        def out_copy(r0):
            pltpu.sync_copy(acc_sh.at[pl.ds(r0, ZR)], stage_v)
            pltpu.sync_copy(stage_v, out_hbm.at[c, pl.ds(r0, ZR)])

        _for_my_chunks(s, out_copy)

    return deg_kernel


def _make_agg_kernel(d: int):
    @functools.partial(
        pl.kernel,
        out_type=jax.ShapeDtypeStruct((2, N_NODES, d), jnp.float32),
        mesh=_sc_mesh(),
        scratch_types=[
            pltpu.VMEM((NCHUNK, CH), jnp.int32),
            pltpu.VMEM((NCHUNK, CH), jnp.int32),
            pltpu.VMEM((CH, d), jnp.float32),
            pltpu.VMEM((ZR, d), jnp.float32),
            pltpu.VMEM_SHARED((N_NODES, d), jnp.float32),
            pltpu.SemaphoreType.DMA,
        ],
        compiler_params=pltpu.CompilerParams(use_tc_tiling_on_sc=False),
    )
    def agg_kernel(y_hbm, src_hbm, dst_hbm, out_hbm,
                   src_v, dst_v, rows_v, stage_v, acc_sh, sem):
        c = lax.axis_index("c")
        s = lax.axis_index("s")
        wid = c * 16 + s

        _zero_fill(stage_v, ZR, d)
        _for_my_chunks(s, lambda r0: pltpu.sync_copy(
            stage_v, acc_sh.at[pl.ds(r0, ZR)]))

        pltpu.sync_copy(src_hbm.at[wid], src_v)
        pltpu.sync_copy(dst_hbm.at[wid], dst_v)
        plsc.subcore_barrier()

        def body(j, carry):
            pltpu.async_copy(y_hbm.at[src_v.at[j]], rows_v, sem).wait()
            pltpu.sync_copy(rows_v, acc_sh.at[dst_v.at[j]], add=True)
            return carry

        lax.fori_loop(0, NCHUNK, body, 0)
        plsc.subcore_barrier()

        def out_copy(r0):
            pltpu.sync_copy(acc_sh.at[pl.ds(r0, ZR)], stage_v)
            pltpu.sync_copy(stage_v, out_hbm.at[c, pl.ds(r0, ZR)])

        _for_my_chunks(s, out_copy)

    return agg_kernel


_deg_call = _make_deg_kernel()
_agg64_call = _make_agg_kernel(D2PAD)


_ROWS = 2000


def _dinv_block(degp_block):
    deg = degp_block[0, :, :1] + degp_block[1, :, :1] + 1.0
    return lax.rsqrt(deg)


def _tc1_body(x_ref, w1_ref, degp_ref, y1a_ref, y1b_ref):
    dinv = _dinv_block(degp_ref[...])
    xw = jnp.dot(x_ref[...], w1_ref[...], preferred_element_type=jnp.float32)
    y = xw * dinv
    y1a_ref[...] = y[:, :D2PAD]
    y1b_ref[...] = y[:, D2PAD:]


def _tc2_body(p1a_ref, p1b_ref, y1a_ref, y1b_ref, degp_ref, w2p_ref, b1_ref,
              y2p_ref):
    dinv = _dinv_block(degp_ref[...])
    pa = p1a_ref[...]
    pb = p1b_ref[...]
    agg = jnp.concatenate(
        [pa[0] + pa[1] + y1a_ref[...], pb[0] + pb[1] + y1b_ref[...]], axis=1)
    h = jnp.maximum(agg * dinv + b1_ref[...], 0.0)
    hw = jnp.dot(h, w2p_ref[...], preferred_element_type=jnp.float32)
    y2p_ref[...] = hw * dinv


def _tc3_body(p2_ref, y2p_ref, degp_ref, b2_ref, out_ref):
    dinv = _dinv_block(degp_ref[...])
    p = p2_ref[...]
    agg = (p[0] + p[1] + y2p_ref[...])[:, :NCLASS]
    logits = agg * dinv + b2_ref[...]
    m = jnp.max(logits, axis=1, keepdims=True)
    sh = logits - m
    lse = jnp.log(jnp.sum(jnp.exp(sh), axis=1, keepdims=True))
    out_ref[...] = sh - lse


def _row_spec(d):
    return pl.BlockSpec((_ROWS, d), lambda i: (i, 0))


def _part_spec(d):
    return pl.BlockSpec((2, _ROWS, d), lambda i: (0, i, 0))


def _full_spec(r, d):
    return pl.BlockSpec((r, d), lambda i: (0, 0))


_GRID = N_NODES // _ROWS

_tc1_call = pl.pallas_call(
    _tc1_body,
    grid=(_GRID,),
    in_specs=[_row_spec(D_IN), _full_spec(D_IN, NHID), _part_spec(16)],
    out_specs=[_row_spec(D2PAD), _row_spec(D2PAD)],
    out_shape=[jax.ShapeDtypeStruct((N_NODES, D2PAD), jnp.float32),
               jax.ShapeDtypeStruct((N_NODES, D2PAD), jnp.float32)],
)

_tc2_call = pl.pallas_call(
    _tc2_body,
    grid=(_GRID,),
    in_specs=[_part_spec(D2PAD), _part_spec(D2PAD), _row_spec(D2PAD),
              _row_spec(D2PAD), _part_spec(16),
              _full_spec(NHID, D2PAD), _full_spec(1, NHID)],
    out_specs=_row_spec(D2PAD),
    out_shape=jax.ShapeDtypeStruct((N_NODES, D2PAD), jnp.float32),
)

_tc3_call = pl.pallas_call(
    _tc3_body,
    grid=(_GRID,),
    in_specs=[_part_spec(D2PAD), _row_spec(D2PAD), _part_spec(16),
              _full_spec(1, NCLASS)],
    out_specs=_row_spec(NCLASS),
    out_shape=jax.ShapeDtypeStruct((N_NODES, NCLASS), jnp.float32),
)


@jax.jit
def kernel(x, edge_index, W1, b1, W2, b2):
    src3 = edge_index[0].reshape(NW, NCHUNK, CH)
    dst3 = edge_index[1].reshape(NW, NCHUNK, CH)

    degp = _deg_call(dst3)
    y1a, y1b = _tc1_call(x, W1, degp)
    p1a = _agg64_call(y1a, src3, dst3)
    p1b = _agg64_call(y1b, src3, dst3)
    w2p = jnp.pad(W2, ((0, 0), (0, D2PAD - NCLASS)))
    y2p = _tc2_call(p1a, p1b, y1a, y1b, degp, w2p, b1.reshape(1, NHID))
    p2 = _agg64_call(y2p, src3, dst3)
    return _tc3_call(p2, y2p, degp, b2.reshape(1, NCLASS))


# speedup vs baseline: 20.4772x; 1.1857x over previous
import functools

import jax
import jax.numpy as jnp
from jax import lax
from jax.experimental import pallas as pl
from jax.experimental.pallas import tpu as pltpu
import jax.experimental.pallas.tpu_sc as plsc

N_NODES = 10000
N_EDGES = 320000
D_IN = 128
NHID = 128
NCLASS = 40
D2PAD = 64

NW = 32
EPW = N_EDGES // NW
CH = 80
NCHUNK = EPW // CH
ZR = 200
NZCH = N_NODES // ZR


def _sc_mesh():
    return plsc.VectorSubcoreMesh(core_axis_name="c", subcore_axis_name="s")


def _for_my_chunks(s, fn):
    def body(i, carry):
        k = s + 16 * i

        @pl.when(k < NZCH)
        def _():
            fn(k * ZR)

        return carry

    lax.fori_loop(0, (NZCH + 15) // 16, body, 0)


def _zero_fill(buf, rows, cols):
    nvec = rows * (cols // 16)

    def body(i, carry):
        r = i // (cols // 16)
        k = i % (cols // 16)
        buf[r, pl.ds(k * 16, 16)] = jnp.zeros((16,), jnp.float32)
        return carry

    lax.fori_loop(0, nvec, body, 0)


def _make_deg_kernel():
    @functools.partial(
        pl.kernel,
        out_type=jax.ShapeDtypeStruct((2, N_NODES, 16), jnp.float32),
        mesh=_sc_mesh(),
        scratch_types=[
            pltpu.VMEM((NCHUNK, CH), jnp.int32),
            pltpu.VMEM((CH, 16), jnp.float32),
            pltpu.VMEM((ZR, 16), jnp.float32),
            pltpu.VMEM_SHARED((N_NODES, 16), jnp.float32),
        ],
        compiler_params=pltpu.CompilerParams(use_tc_tiling_on_sc=False),
    )
    def deg_kernel(dst_hbm, out_hbm, idx_v, ones_v, stage_v, acc_sh):
        c = lax.axis_index("c")
        s = lax.axis_index("s")
        wid = c * 16 + s

        _zero_fill(stage_v, ZR, 16)
        _for_my_chunks(s, lambda r0: pltpu.sync_copy(
            stage_v, acc_sh.at[pl.ds(r0, ZR)]))

        def ones_fill(i, carry):
            ones_v[i, :] = jnp.ones((16,), jnp.float32)
            return carry

        lax.fori_loop(0, CH, ones_fill, 0)

        pltpu.sync_copy(dst_hbm.at[wid], idx_v)
        plsc.subcore_barrier()

        def body(j, carry):
            pltpu.sync_copy(ones_v, acc_sh.at[idx_v.at[j]], add=True)
            return carry

        lax.fori_loop(0, NCHUNK, body, 0)
        plsc.subcore_barrier()

        def out_copy(r0):
            pltpu.sync_copy(acc_sh.at[pl.ds(r0, ZR)], stage_v)
            pltpu.sync_copy(stage_v, out_hbm.at[c, pl.ds(r0, ZR)])

        _for_my_chunks(s, out_copy)

    return deg_kernel


def _make_agg_kernel(d: int):
    @functools.partial(
        pl.kernel,
        out_type=jax.ShapeDtypeStruct((2, N_NODES, d), jnp.float32),
        mesh=_sc_mesh(),
        scratch_types=[
            pltpu.VMEM((NCHUNK, CH), jnp.int32),
            pltpu.VMEM((NCHUNK, CH), jnp.int32),
            pltpu.VMEM((CH, d), jnp.float32),
            pltpu.VMEM((CH, d), jnp.float32),
            pltpu.VMEM((ZR, d), jnp.float32),
            pltpu.VMEM_SHARED((N_NODES, d), jnp.float32),
            pltpu.SemaphoreType.DMA,
            pltpu.SemaphoreType.DMA,
            pltpu.SemaphoreType.DMA,
            pltpu.SemaphoreType.DMA,
        ],
        compiler_params=pltpu.CompilerParams(use_tc_tiling_on_sc=False),
    )
    def agg_kernel(y_hbm, src_hbm, dst_hbm, out_hbm,
                   src_v, dst_v, rows_a, rows_b, stage_v, acc_sh,
                   sga, sgb, ssa, ssb):
        c = lax.axis_index("c")
        s = lax.axis_index("s")
        wid = c * 16 + s

        _zero_fill(stage_v, ZR, d)
        _for_my_chunks(s, lambda r0: pltpu.sync_copy(
            stage_v, acc_sh.at[pl.ds(r0, ZR)]))

        pltpu.sync_copy(src_hbm.at[wid], src_v)
        pltpu.sync_copy(dst_hbm.at[wid], dst_v)
        plsc.subcore_barrier()

        def gather(j, buf, sem):
            pltpu.async_copy(y_hbm.at[src_v.at[j]], buf, sem)

        def scatter(j, buf, sem):
            pltpu.async_copy(buf, acc_sh.at[dst_v.at[j]], sem, add=True)

        def wait_g(buf, sem):
            pltpu.make_async_copy(y_hbm.at[src_v.at[0]], buf, sem).wait()

        def wait_s(buf, sem):
            pltpu.make_async_copy(buf, acc_sh.at[dst_v.at[0]], sem).wait()

        gather(0, rows_a, sga)

        def body(i, carry):
            ja = 2 * i

            @pl.when(i > 0)
            def _():
                wait_s(rows_b, ssb)

            wait_g(rows_a, sga)
            gather(ja + 1, rows_b, sgb)
            scatter(ja, rows_a, ssa)
            wait_g(rows_b, sgb)
            wait_s(rows_a, ssa)
            gather(ja + 2, rows_a, sga)
            scatter(ja + 1, rows_b, ssb)
            return carry

        lax.fori_loop(0, (NCHUNK - 1) // 2, body, 0)
        wait_s(rows_b, ssb)
        wait_g(rows_a, sga)
        scatter(NCHUNK - 1, rows_a, ssa)
        wait_s(rows_a, ssa)
        plsc.subcore_barrier()

        def out_copy(r0):
            pltpu.sync_copy(acc_sh.at[pl.ds(r0, ZR)], stage_v)
            pltpu.sync_copy(stage_v, out_hbm.at[c, pl.ds(r0, ZR)])

        _for_my_chunks(s, out_copy)

    return agg_kernel


_deg_call = _make_deg_kernel()
_agg64_call = _make_agg_kernel(D2PAD)


_ROWS = 2000


def _dinv_block(degp_block):
    deg = degp_block[0, :, :1] + degp_block[1, :, :1] + 1.0
    return lax.rsqrt(deg)


def _tc1_body(x_ref, w1_ref, degp_ref, y1a_ref, y1b_ref):
    dinv = _dinv_block(degp_ref[...])
    xw = jnp.dot(x_ref[...], w1_ref[...], preferred_element_type=jnp.float32)
    y = xw * dinv
    y1a_ref[...] = y[:, :D2PAD]
    y1b_ref[...] = y[:, D2PAD:]


def _tc2_body(p1a_ref, p1b_ref, y1a_ref, y1b_ref, degp_ref, w2p_ref, b1_ref,
              y2p_ref):
    dinv = _dinv_block(degp_ref[...])
    pa = p1a_ref[...]
    pb = p1b_ref[...]
    agg = jnp.concatenate(
        [pa[0] + pa[1] + y1a_ref[...], pb[0] + pb[1] + y1b_ref[...]], axis=1)
    h = jnp.maximum(agg * dinv + b1_ref[...], 0.0)
    hw = jnp.dot(h, w2p_ref[...], preferred_element_type=jnp.float32)
    y2p_ref[...] = hw * dinv


def _tc3_body(p2_ref, y2p_ref, degp_ref, b2_ref, out_ref):
    dinv = _dinv_block(degp_ref[...])
    p = p2_ref[...]
    agg = (p[0] + p[1] + y2p_ref[...])[:, :NCLASS]
    logits = agg * dinv + b2_ref[...]
    m = jnp.max(logits, axis=1, keepdims=True)
    sh = logits - m
    lse = jnp.log(jnp.sum(jnp.exp(sh), axis=1, keepdims=True))
    out_ref[...] = sh - lse


def _row_spec(d):
    return pl.BlockSpec((_ROWS, d), lambda i: (i, 0))


def _part_spec(d):
    return pl.BlockSpec((2, _ROWS, d), lambda i: (0, i, 0))


def _full_spec(r, d):
    return pl.BlockSpec((r, d), lambda i: (0, 0))


_GRID = N_NODES // _ROWS

_tc1_call = pl.pallas_call(
    _tc1_body,
    grid=(_GRID,),
    in_specs=[_row_spec(D_IN), _full_spec(D_IN, NHID), _part_spec(16)],
    out_specs=[_row_spec(D2PAD), _row_spec(D2PAD)],
    out_shape=[jax.ShapeDtypeStruct((N_NODES, D2PAD), jnp.float32),
               jax.ShapeDtypeStruct((N_NODES, D2PAD), jnp.float32)],
)

_tc2_call = pl.pallas_call(
    _tc2_body,
    grid=(_GRID,),
    in_specs=[_part_spec(D2PAD), _part_spec(D2PAD), _row_spec(D2PAD),
              _row_spec(D2PAD), _part_spec(16),
              _full_spec(NHID, D2PAD), _full_spec(1, NHID)],
    out_specs=_row_spec(D2PAD),
    out_shape=jax.ShapeDtypeStruct((N_NODES, D2PAD), jnp.float32),
)

_tc3_call = pl.pallas_call(
    _tc3_body,
    grid=(_GRID,),
    in_specs=[_part_spec(D2PAD), _row_spec(D2PAD), _part_spec(16),
              _full_spec(1, NCLASS)],
    out_specs=_row_spec(NCLASS),
    out_shape=jax.ShapeDtypeStruct((N_NODES, NCLASS), jnp.float32),
)


@jax.jit
def kernel(x, edge_index, W1, b1, W2, b2):
    src3 = edge_index[0].reshape(NW, NCHUNK, CH)
    dst3 = edge_index[1].reshape(NW, NCHUNK, CH)

    degp = _deg_call(dst3)
    y1a, y1b = _tc1_call(x, W1, degp)
    p1a = _agg64_call(y1a, src3, dst3)
    p1b = _agg64_call(y1b, src3, dst3)
    w2p = jnp.pad(W2, ((0, 0), (0, D2PAD - NCLASS)))
    y2p = _tc2_call(p1a, p1b, y1a, y1b, degp, w2p, b1.reshape(1, NHID))
    p2 = _agg64_call(y2p, src3, dst3)
    return _tc3_call(p2, y2p, degp, b2.reshape(1, NCLASS))


# speedup vs baseline: 24.5368x; 1.1982x over previous
import functools

import jax
import jax.numpy as jnp
from jax import lax
from jax.experimental import pallas as pl
from jax.experimental.pallas import tpu as pltpu
import jax.experimental.pallas.tpu_sc as plsc

N_NODES = 10000
N_EDGES = 320000
D_IN = 128
NHID = 128
NCLASS = 40
D2PAD = 64

NW = 32
CH = 128
NCHUNK = -(-N_EDGES // (NW * CH))
EPW = NCHUNK * CH
PADE = NW * EPW - N_EDGES
ACC_PAD = 8
ACC_ROWS = N_NODES + ACC_PAD
ZR = 200
NZCH = N_NODES // ZR


def _sc_mesh():
    return plsc.VectorSubcoreMesh(core_axis_name="c", subcore_axis_name="s")


def _for_my_chunks(s, fn):
    def body(i, carry):
        k = s + 16 * i

        @pl.when(k < NZCH)
        def _():
            fn(k * ZR)

        return carry

    lax.fori_loop(0, (NZCH + 15) // 16, body, 0)


def _zero_fill(buf, rows, cols):
    nvec = rows * (cols // 16)

    def body(i, carry):
        r = i // (cols // 16)
        k = i % (cols // 16)
        buf[r, pl.ds(k * 16, 16)] = jnp.zeros((16,), jnp.float32)
        return carry

    lax.fori_loop(0, nvec, body, 0)


def _make_deg_kernel():
    @functools.partial(
        pl.kernel,
        out_type=jax.ShapeDtypeStruct((2, N_NODES, 16), jnp.float32),
        mesh=_sc_mesh(),
        scratch_types=[
            pltpu.VMEM((NCHUNK, CH), jnp.int32),
            pltpu.VMEM((CH, 16), jnp.float32),
            pltpu.VMEM((ZR, 16), jnp.float32),
            pltpu.VMEM_SHARED((ACC_ROWS, 16), jnp.float32),
        ],
        compiler_params=pltpu.CompilerParams(use_tc_tiling_on_sc=False),
    )
    def deg_kernel(dst_hbm, out_hbm, idx_v, ones_v, stage_v, acc_sh):
        c = lax.axis_index("c")
        s = lax.axis_index("s")
        wid = c * 16 + s

        _zero_fill(stage_v, ZR, 16)
        _for_my_chunks(s, lambda r0: pltpu.sync_copy(
            stage_v, acc_sh.at[pl.ds(r0, ZR)]))

        def ones_fill(i, carry):
            ones_v[i, :] = jnp.ones((16,), jnp.float32)
            return carry

        lax.fori_loop(0, CH, ones_fill, 0)

        pltpu.sync_copy(dst_hbm.at[wid], idx_v)
        plsc.subcore_barrier()

        def body(j, carry):
            pltpu.sync_copy(ones_v, acc_sh.at[idx_v.at[j]], add=True)
            return carry

        lax.fori_loop(0, NCHUNK, body, 0)
        plsc.subcore_barrier()

        def out_copy(r0):
            pltpu.sync_copy(acc_sh.at[pl.ds(r0, ZR)], stage_v)
            pltpu.sync_copy(stage_v, out_hbm.at[c, pl.ds(r0, ZR)])

        _for_my_chunks(s, out_copy)

    return deg_kernel


def _make_agg_kernel(d: int):
    @functools.partial(
        pl.kernel,
        out_type=jax.ShapeDtypeStruct((2, N_NODES, d), jnp.float32),
        mesh=_sc_mesh(),
        scratch_types=[
            pltpu.VMEM((NCHUNK, CH), jnp.int32),
            pltpu.VMEM((NCHUNK, CH), jnp.int32),
            pltpu.VMEM((CH, d), jnp.float32),
            pltpu.VMEM((CH, d), jnp.float32),
            pltpu.VMEM((ZR, d), jnp.float32),
            pltpu.VMEM_SHARED((ACC_ROWS, d), jnp.float32),
            pltpu.SemaphoreType.DMA,
            pltpu.SemaphoreType.DMA,
            pltpu.SemaphoreType.DMA,
            pltpu.SemaphoreType.DMA,
        ],
        compiler_params=pltpu.CompilerParams(use_tc_tiling_on_sc=False),
    )
    def agg_kernel(y_hbm, src_hbm, dst_hbm, out_hbm,
                   src_v, dst_v, rows_a, rows_b, stage_v, acc_sh,
                   sga, sgb, ssa, ssb):
        c = lax.axis_index("c")
        s = lax.axis_index("s")
        wid = c * 16 + s

        _zero_fill(stage_v, ZR, d)
        _for_my_chunks(s, lambda r0: pltpu.sync_copy(
            stage_v, acc_sh.at[pl.ds(r0, ZR)]))

        pltpu.sync_copy(src_hbm.at[wid], src_v)
        pltpu.sync_copy(dst_hbm.at[wid], dst_v)
        plsc.subcore_barrier()

        def gather(j, buf, sem):
            pltpu.async_copy(y_hbm.at[src_v.at[j]], buf, sem)

        def scatter(j, buf, sem):
            pltpu.async_copy(buf, acc_sh.at[dst_v.at[j]], sem, add=True)

        def wait_g(buf, sem):
            pltpu.make_async_copy(y_hbm.at[src_v.at[0]], buf, sem).wait()

        def wait_s(buf, sem):
            pltpu.make_async_copy(buf, acc_sh.at[dst_v.at[0]], sem).wait()

        gather(0, rows_a, sga)

        def body(i, carry):
            ja = 2 * i

            @pl.when(i > 0)
            def _():
                wait_s(rows_b, ssb)

            wait_g(rows_a, sga)
            gather(ja + 1, rows_b, sgb)
            scatter(ja, rows_a, ssa)
            wait_g(rows_b, sgb)
            wait_s(rows_a, ssa)
            gather(ja + 2, rows_a, sga)
            scatter(ja + 1, rows_b, ssb)
            return carry

        lax.fori_loop(0, (NCHUNK - 1) // 2, body, 0)
        wait_s(rows_b, ssb)
        wait_g(rows_a, sga)
        scatter(NCHUNK - 1, rows_a, ssa)
        wait_s(rows_a, ssa)
        plsc.subcore_barrier()

        def out_copy(r0):
            pltpu.sync_copy(acc_sh.at[pl.ds(r0, ZR)], stage_v)
            pltpu.sync_copy(stage_v, out_hbm.at[c, pl.ds(r0, ZR)])

        _for_my_chunks(s, out_copy)

    return agg_kernel


_deg_call = _make_deg_kernel()
_agg64_call = _make_agg_kernel(D2PAD)


_ROWS = 2000


def _dinv_block(degp_block):
    deg = degp_block[0, :, :1] + degp_block[1, :, :1] + 1.0
    return lax.rsqrt(deg)


def _tc1_body(x_ref, w1_ref, degp_ref, y1a_ref, y1b_ref):
    dinv = _dinv_block(degp_ref[...])
    xw = jnp.dot(x_ref[...], w1_ref[...], preferred_element_type=jnp.float32)
    y = xw * dinv
    y1a_ref[...] = y[:, :D2PAD]
    y1b_ref[...] = y[:, D2PAD:]


def _tc2_body(p1a_ref, p1b_ref, y1a_ref, y1b_ref, degp_ref, w2p_ref, b1_ref,
              y2p_ref):
    dinv = _dinv_block(degp_ref[...])
    pa = p1a_ref[...]
    pb = p1b_ref[...]
    agg = jnp.concatenate(
        [pa[0] + pa[1] + y1a_ref[...], pb[0] + pb[1] + y1b_ref[...]], axis=1)
    h = jnp.maximum(agg * dinv + b1_ref[...], 0.0)
    hw = jnp.dot(h, w2p_ref[...], preferred_element_type=jnp.float32)
    y2p_ref[...] = hw * dinv


def _tc3_body(p2_ref, y2p_ref, degp_ref, b2_ref, out_ref):
    dinv = _dinv_block(degp_ref[...])
    p = p2_ref[...]
    agg = (p[0] + p[1] + y2p_ref[...])[:, :NCLASS]
    logits = agg * dinv + b2_ref[...]
    m = jnp.max(logits, axis=1, keepdims=True)
    sh = logits - m
    lse = jnp.log(jnp.sum(jnp.exp(sh), axis=1, keepdims=True))
    out_ref[...] = sh - lse


def _row_spec(d):
    return pl.BlockSpec((_ROWS, d), lambda i: (i, 0))


def _part_spec(d):
    return pl.BlockSpec((2, _ROWS, d), lambda i: (0, i, 0))


def _full_spec(r, d):
    return pl.BlockSpec((r, d), lambda i: (0, 0))


_GRID = N_NODES // _ROWS

_tc1_call = pl.pallas_call(
    _tc1_body,
    grid=(_GRID,),
    in_specs=[_row_spec(D_IN), _full_spec(D_IN, NHID), _part_spec(16)],
    out_specs=[_row_spec(D2PAD), _row_spec(D2PAD)],
    out_shape=[jax.ShapeDtypeStruct((N_NODES, D2PAD), jnp.float32),
               jax.ShapeDtypeStruct((N_NODES, D2PAD), jnp.float32)],
)

_tc2_call = pl.pallas_call(
    _tc2_body,
    grid=(_GRID,),
    in_specs=[_part_spec(D2PAD), _part_spec(D2PAD), _row_spec(D2PAD),
              _row_spec(D2PAD), _part_spec(16),
              _full_spec(NHID, D2PAD), _full_spec(1, NHID)],
    out_specs=_row_spec(D2PAD),
    out_shape=jax.ShapeDtypeStruct((N_NODES, D2PAD), jnp.float32),
)

_tc3_call = pl.pallas_call(
    _tc3_body,
    grid=(_GRID,),
    in_specs=[_part_spec(D2PAD), _row_spec(D2PAD), _part_spec(16),
              _full_spec(1, NCLASS)],
    out_specs=_row_spec(NCLASS),
    out_shape=jax.ShapeDtypeStruct((N_NODES, NCLASS), jnp.float32),
)


@jax.jit
def kernel(x, edge_index, W1, b1, W2, b2):
    pad_src = (jnp.arange(PADE, dtype=jnp.int32) * 97) % N_NODES
    pad_dst = N_NODES + (jnp.arange(PADE, dtype=jnp.int32) % ACC_PAD)
    src3 = jnp.concatenate([edge_index[0], pad_src]).reshape(NW, NCHUNK, CH)
    dst3 = jnp.concatenate([edge_index[1], pad_dst]).reshape(NW, NCHUNK, CH)

    degp = _deg_call(dst3)
    y1a, y1b = _tc1_call(x, W1, degp)
    p1a = _agg64_call(y1a, src3, dst3)
    p1b = _agg64_call(y1b, src3, dst3)
    w2p = jnp.pad(W2, ((0, 0), (0, D2PAD - NCLASS)))
    y2p = _tc2_call(p1a, p1b, y1a, y1b, degp, w2p, b1.reshape(1, NHID))
    p2 = _agg64_call(y2p, src3, dst3)
    return _tc3_call(p2, y2p, degp, b2.reshape(1, NCLASS))


# speedup vs baseline: 25.5837x; 1.0427x over previous
import functools

import jax
import jax.numpy as jnp
from jax import lax
from jax.experimental import pallas as pl
from jax.experimental.pallas import tpu as pltpu
import jax.experimental.pallas.tpu_sc as plsc

N_NODES = 10000
N_EDGES = 320000
D_IN = 128
NHID = 128
NCLASS = 40
D2PAD = 64

NW = 32
CH = 128
NCHUNK = -(-N_EDGES // (NW * CH))
EPW = NCHUNK * CH
PADE = NW * EPW - N_EDGES
ACC_PAD = 8
ACC_ROWS = N_NODES + ACC_PAD
ZR = 200
NZCH = N_NODES // ZR


def _sc_mesh():
    return plsc.VectorSubcoreMesh(core_axis_name="c", subcore_axis_name="s")


def _for_my_chunks(s, fn):
    def body(i, carry):
        k = s + 16 * i

        @pl.when(k < NZCH)
        def _():
            fn(k * ZR)

        return carry

    lax.fori_loop(0, (NZCH + 15) // 16, body, 0)


def _pipeline(nchunk, gather, scatter, wait_g, wait_s,
              rows_a, rows_b, sga, sgb, ssa, ssb):
    even = nchunk % 2 == 0
    pairs = (nchunk - 2) // 2 if even else (nchunk - 1) // 2
    gather(0, rows_a, sga)

    def body(i, carry):
        ja = 2 * i

        @pl.when(i > 0)
        def _():
            wait_s(rows_b, ssb)

        wait_g(rows_a, sga)
        gather(ja + 1, rows_b, sgb)
        scatter(ja, rows_a, ssa)
        wait_g(rows_b, sgb)
        wait_s(rows_a, ssa)
        gather(ja + 2, rows_a, sga)
        scatter(ja + 1, rows_b, ssb)
        return carry

    lax.fori_loop(0, pairs, body, 0)
    if pairs > 0:
        wait_s(rows_b, ssb)
    wait_g(rows_a, sga)
    if even:
        gather(nchunk - 1, rows_b, sgb)
        scatter(nchunk - 2, rows_a, ssa)
        wait_g(rows_b, sgb)
        wait_s(rows_a, ssa)
        scatter(nchunk - 1, rows_b, ssb)
        wait_s(rows_b, ssb)
    else:
        scatter(nchunk - 1, rows_a, ssa)
        wait_s(rows_a, ssa)


def _zero_fill(buf, rows, cols):
    nvec = rows * (cols // 16)

    def body(i, carry):
        r = i // (cols // 16)
        k = i % (cols // 16)
        buf[r, pl.ds(k * 16, 16)] = jnp.zeros((16,), jnp.float32)
        return carry

    lax.fori_loop(0, nvec, body, 0)


def _make_deg_kernel():
    @functools.partial(
        pl.kernel,
        out_type=jax.ShapeDtypeStruct((2, N_NODES, 16), jnp.float32),
        mesh=_sc_mesh(),
        scratch_types=[
            pltpu.VMEM((NCHUNK, CH), jnp.int32),
            pltpu.VMEM((CH, 16), jnp.float32),
            pltpu.VMEM((ZR, 16), jnp.float32),
            pltpu.VMEM_SHARED((ACC_ROWS, 16), jnp.float32),
        ],
        compiler_params=pltpu.CompilerParams(use_tc_tiling_on_sc=False),
    )
    def deg_kernel(dst_hbm, out_hbm, idx_v, ones_v, stage_v, acc_sh):
        c = lax.axis_index("c")
        s = lax.axis_index("s")
        wid = c * 16 + s

        _zero_fill(stage_v, ZR, 16)
        _for_my_chunks(s, lambda r0: pltpu.sync_copy(
            stage_v, acc_sh.at[pl.ds(r0, ZR)]))

        def ones_fill(i, carry):
            ones_v[i, :] = jnp.ones((16,), jnp.float32)
            return carry

        lax.fori_loop(0, CH, ones_fill, 0)

        pltpu.sync_copy(dst_hbm.at[wid], idx_v)
        plsc.subcore_barrier()

        def body(j, carry):
            pltpu.sync_copy(ones_v, acc_sh.at[idx_v.at[j]], add=True)
            return carry

        lax.fori_loop(0, NCHUNK, body, 0)
        plsc.subcore_barrier()

        def out_copy(r0):
            pltpu.sync_copy(acc_sh.at[pl.ds(r0, ZR)], stage_v)
            pltpu.sync_copy(stage_v, out_hbm.at[c, pl.ds(r0, ZR)])

        _for_my_chunks(s, out_copy)

    return deg_kernel


def _make_agg_kernel(d: int):
    @functools.partial(
        pl.kernel,
        out_type=jax.ShapeDtypeStruct((2, N_NODES, d), jnp.float32),
        mesh=_sc_mesh(),
        scratch_types=[
            pltpu.VMEM((NCHUNK, CH), jnp.int32),
            pltpu.VMEM((NCHUNK, CH), jnp.int32),
            pltpu.VMEM((CH, d), jnp.float32),
            pltpu.VMEM((CH, d), jnp.float32),
            pltpu.VMEM((ZR, d), jnp.float32),
            pltpu.VMEM_SHARED((ACC_ROWS, d), jnp.float32),
            pltpu.SemaphoreType.DMA,
            pltpu.SemaphoreType.DMA,
            pltpu.SemaphoreType.DMA,
            pltpu.SemaphoreType.DMA,
        ],
        compiler_params=pltpu.CompilerParams(use_tc_tiling_on_sc=False),
    )
    def agg_kernel(y_hbm, src_hbm, dst_hbm, out_hbm,
                   src_v, dst_v, rows_a, rows_b, stage_v, acc_sh,
                   sga, sgb, ssa, ssb):
        c = lax.axis_index("c")
        s = lax.axis_index("s")
        wid = c * 16 + s

        _zero_fill(stage_v, ZR, d)
        _for_my_chunks(s, lambda r0: pltpu.sync_copy(
            stage_v, acc_sh.at[pl.ds(r0, ZR)]))

        pltpu.sync_copy(src_hbm.at[wid], src_v)
        pltpu.sync_copy(dst_hbm.at[wid], dst_v)
        plsc.subcore_barrier()

        def gather(j, buf, sem):
            pltpu.async_copy(y_hbm.at[src_v.at[j]], buf, sem)

        def scatter(j, buf, sem):
            pltpu.async_copy(buf, acc_sh.at[dst_v.at[j]], sem, add=True)

        def wait_g(buf, sem):
            pltpu.make_async_copy(y_hbm.at[src_v.at[0]], buf, sem).wait()

        def wait_s(buf, sem):
            pltpu.make_async_copy(buf, acc_sh.at[dst_v.at[0]], sem).wait()

        _pipeline(NCHUNK, gather, scatter, wait_g, wait_s,
                  rows_a, rows_b, sga, sgb, ssa, ssb)
        plsc.subcore_barrier()

        def out_copy(r0):
            pltpu.sync_copy(acc_sh.at[pl.ds(r0, ZR)], stage_v)
            pltpu.sync_copy(stage_v, out_hbm.at[c, pl.ds(r0, ZR)])

        _for_my_chunks(s, out_copy)

    return agg_kernel


NT2 = 16
NCHUNK2 = NW * NCHUNK // NT2


def _make_agg_dual_kernel():
    d = D2PAD

    @functools.partial(
        pl.kernel,
        out_type=jax.ShapeDtypeStruct((2, N_NODES, d), jnp.float32),
        mesh=_sc_mesh(),
        scratch_types=[
            pltpu.VMEM((NCHUNK2, CH), jnp.int32),
            pltpu.VMEM((NCHUNK2, CH), jnp.int32),
            pltpu.VMEM((CH, d), jnp.float32),
            pltpu.VMEM((CH, d), jnp.float32),
            pltpu.VMEM((ZR, d), jnp.float32),
            pltpu.VMEM_SHARED((ACC_ROWS, d), jnp.float32),
            pltpu.SemaphoreType.DMA,
            pltpu.SemaphoreType.DMA,
            pltpu.SemaphoreType.DMA,
            pltpu.SemaphoreType.DMA,
        ],
        compiler_params=pltpu.CompilerParams(use_tc_tiling_on_sc=False),
    )
    def agg_dual_kernel(ya_hbm, yb_hbm, src_hbm, dst_hbm, out_hbm,
                        src_v, dst_v, rows_a, rows_b, stage_v, acc_sh,
                        sga, sgb, ssa, ssb):
        c = lax.axis_index("c")
        s = lax.axis_index("s")

        _zero_fill(stage_v, ZR, d)
        _for_my_chunks(s, lambda r0: pltpu.sync_copy(
            stage_v, acc_sh.at[pl.ds(r0, ZR)]))

        pltpu.sync_copy(src_hbm.at[s], src_v)
        pltpu.sync_copy(dst_hbm.at[s], dst_v)
        plsc.subcore_barrier()

        def run(y_hbm):
            def gather(j, buf, sem):
                pltpu.async_copy(y_hbm.at[src_v.at[j]], buf, sem)

            def scatter(j, buf, sem):
                pltpu.async_copy(buf, acc_sh.at[dst_v.at[j]], sem, add=True)

            def wait_g(buf, sem):
                pltpu.make_async_copy(y_hbm.at[src_v.at[0]], buf, sem).wait()

            def wait_s(buf, sem):
                pltpu.make_async_copy(buf, acc_sh.at[dst_v.at[0]], sem).wait()

            _pipeline(NCHUNK2, gather, scatter, wait_g, wait_s,
                      rows_a, rows_b, sga, sgb, ssa, ssb)

        @pl.when(c == 0)
        def _():
            run(ya_hbm)

        @pl.when(c == 1)
        def _():
            run(yb_hbm)

        plsc.subcore_barrier()

        def out_copy(r0):
            pltpu.sync_copy(acc_sh.at[pl.ds(r0, ZR)], stage_v)
            pltpu.sync_copy(stage_v, out_hbm.at[c, pl.ds(r0, ZR)])

        _for_my_chunks(s, out_copy)

    return agg_dual_kernel


_deg_call = _make_deg_kernel()
_agg_dual_call = _make_agg_dual_kernel()
_agg64_call = _make_agg_kernel(D2PAD)


_ROWS = 2000


def _dinv_block(degp_block):
    deg = degp_block[0, :, :1] + degp_block[1, :, :1] + 1.0
    return lax.rsqrt(deg)


def _tc1_body(x_ref, w1_ref, degp_ref, y1a_ref, y1b_ref):
    dinv = _dinv_block(degp_ref[...])
    xw = jnp.dot(x_ref[...], w1_ref[...], preferred_element_type=jnp.float32)
    y = xw * dinv
    y1a_ref[...] = y[:, :D2PAD]
    y1b_ref[...] = y[:, D2PAD:]


def _tc2_body(p1_ref, y1a_ref, y1b_ref, degp_ref, w2p_ref, b1_ref,
              y2p_ref):
    dinv = _dinv_block(degp_ref[...])
    p = p1_ref[...]
    agg = jnp.concatenate(
        [p[0] + y1a_ref[...], p[1] + y1b_ref[...]], axis=1)
    h = jnp.maximum(agg * dinv + b1_ref[...], 0.0)
    hw = jnp.dot(h, w2p_ref[...], preferred_element_type=jnp.float32)
    y2p_ref[...] = hw * dinv


def _tc3_body(p2_ref, y2p_ref, degp_ref, b2_ref, out_ref):
    dinv = _dinv_block(degp_ref[...])
    p = p2_ref[...]
    agg = (p[0] + p[1] + y2p_ref[...])[:, :NCLASS]
    logits = agg * dinv + b2_ref[...]
    m = jnp.max(logits, axis=1, keepdims=True)
    sh = logits - m
    lse = jnp.log(jnp.sum(jnp.exp(sh), axis=1, keepdims=True))
    out_ref[...] = sh - lse


def _row_spec(d):
    return pl.BlockSpec((_ROWS, d), lambda i: (i, 0))


def _part_spec(d):
    return pl.BlockSpec((2, _ROWS, d), lambda i: (0, i, 0))


def _full_spec(r, d):
    return pl.BlockSpec((r, d), lambda i: (0, 0))


_GRID = N_NODES // _ROWS

_tc1_call = pl.pallas_call(
    _tc1_body,
    grid=(_GRID,),
    in_specs=[_row_spec(D_IN), _full_spec(D_IN, NHID), _part_spec(16)],
    out_specs=[_row_spec(D2PAD), _row_spec(D2PAD)],
    out_shape=[jax.ShapeDtypeStruct((N_NODES, D2PAD), jnp.float32),
               jax.ShapeDtypeStruct((N_NODES, D2PAD), jnp.float32)],
)

_tc2_call = pl.pallas_call(
    _tc2_body,
    grid=(_GRID,),
    in_specs=[_part_spec(D2PAD), _row_spec(D2PAD),
              _row_spec(D2PAD), _part_spec(16),
              _full_spec(NHID, D2PAD), _full_spec(1, NHID)],
    out_specs=_row_spec(D2PAD),
    out_shape=jax.ShapeDtypeStruct((N_NODES, D2PAD), jnp.float32),
)

_tc3_call = pl.pallas_call(
    _tc3_body,
    grid=(_GRID,),
    in_specs=[_part_spec(D2PAD), _row_spec(D2PAD), _part_spec(16),
              _full_spec(1, NCLASS)],
    out_specs=_row_spec(NCLASS),
    out_shape=jax.ShapeDtypeStruct((N_NODES, NCLASS), jnp.float32),
)


@jax.jit
def kernel(x, edge_index, W1, b1, W2, b2):
    pad_src = (jnp.arange(PADE, dtype=jnp.int32) * 97) % N_NODES
    pad_dst = N_NODES + (jnp.arange(PADE, dtype=jnp.int32) % ACC_PAD)
    src_p = jnp.concatenate([edge_index[0], pad_src])
    dst_p = jnp.concatenate([edge_index[1], pad_dst])
    src3 = src_p.reshape(NW, NCHUNK, CH)
    dst3 = dst_p.reshape(NW, NCHUNK, CH)
    srcT = src_p.reshape(NT2, NCHUNK2, CH)
    dstT = dst_p.reshape(NT2, NCHUNK2, CH)

    degp = _deg_call(dst3)
    y1a, y1b = _tc1_call(x, W1, degp)
    p1 = _agg_dual_call(y1a, y1b, srcT, dstT)
    w2p = jnp.pad(W2, ((0, 0), (0, D2PAD - NCLASS)))
    y2p = _tc2_call(p1, y1a, y1b, degp, w2p, b1.reshape(1, NHID))
    p2 = _agg64_call(y2p, src3, dst3)
    return _tc3_call(p2, y2p, degp, b2.reshape(1, NCLASS))


# speedup vs baseline: 25.8443x; 1.0102x over previous
import functools

import jax
import jax.numpy as jnp
from jax import lax
from jax.experimental import pallas as pl
from jax.experimental.pallas import tpu as pltpu
import jax.experimental.pallas.tpu_sc as plsc

N_NODES = 10000
N_EDGES = 320000
D_IN = 128
NHID = 128
NCLASS = 40
D2PAD = 64
DCLS = 48

NW = 32
CH = 128
NCHUNK = -(-N_EDGES // (NW * CH))
EPW = NCHUNK * CH
PADE = NW * EPW - N_EDGES
ACC_PAD = 8
ACC_ROWS = N_NODES + ACC_PAD
ZR = 200
NZCH = N_NODES // ZR


def _sc_mesh():
    return plsc.VectorSubcoreMesh(core_axis_name="c", subcore_axis_name="s")


def _for_my_chunks(s, fn):
    def body(i, carry):
        k = s + 16 * i

        @pl.when(k < NZCH)
        def _():
            fn(k * ZR)

        return carry

    lax.fori_loop(0, (NZCH + 15) // 16, body, 0)


def _pipeline(nchunk, gather, scatter, wait_g, wait_s,
              rows_a, rows_b, sga, sgb, ssa, ssb):
    even = nchunk % 2 == 0
    pairs = (nchunk - 2) // 2 if even else (nchunk - 1) // 2
    gather(0, rows_a, sga)

    def body(i, carry):
        ja = 2 * i

        @pl.when(i > 0)
        def _():
            wait_s(rows_b, ssb)

        wait_g(rows_a, sga)
        gather(ja + 1, rows_b, sgb)
        scatter(ja, rows_a, ssa)
        wait_g(rows_b, sgb)
        wait_s(rows_a, ssa)
        gather(ja + 2, rows_a, sga)
        scatter(ja + 1, rows_b, ssb)
        return carry

    lax.fori_loop(0, pairs, body, 0)
    if pairs > 0:
        wait_s(rows_b, ssb)
    wait_g(rows_a, sga)
    if even:
        gather(nchunk - 1, rows_b, sgb)
        scatter(nchunk - 2, rows_a, ssa)
        wait_g(rows_b, sgb)
        wait_s(rows_a, ssa)
        scatter(nchunk - 1, rows_b, ssb)
        wait_s(rows_b, ssb)
    else:
        scatter(nchunk - 1, rows_a, ssa)
        wait_s(rows_a, ssa)


def _zero_fill(buf, rows, cols):
    nvec = rows * (cols // 16)

    def body(i, carry):
        r = i // (cols // 16)
        k = i % (cols // 16)
        buf[r, pl.ds(k * 16, 16)] = jnp.zeros((16,), jnp.float32)
        return carry

    lax.fori_loop(0, nvec, body, 0)


def _make_deg_kernel():
    @functools.partial(
        pl.kernel,
        out_type=jax.ShapeDtypeStruct((2, N_NODES, 16), jnp.float32),
        mesh=_sc_mesh(),
        scratch_types=[
            pltpu.VMEM((NCHUNK, CH), jnp.int32),
            pltpu.VMEM((CH, 16), jnp.float32),
            pltpu.VMEM((ZR, 16), jnp.float32),
            pltpu.VMEM_SHARED((ACC_ROWS, 16), jnp.float32),
        ],
        compiler_params=pltpu.CompilerParams(use_tc_tiling_on_sc=False),
    )
    def deg_kernel(dst_hbm, out_hbm, idx_v, ones_v, stage_v, acc_sh):
        c = lax.axis_index("c")
        s = lax.axis_index("s")
        wid = c * 16 + s

        _zero_fill(stage_v, ZR, 16)
        _for_my_chunks(s, lambda r0: pltpu.sync_copy(
            stage_v, acc_sh.at[pl.ds(r0, ZR)]))

        def ones_fill(i, carry):
            ones_v[i, :] = jnp.ones((16,), jnp.float32)
            return carry

        lax.fori_loop(0, CH, ones_fill, 0)

        pltpu.sync_copy(dst_hbm.at[wid], idx_v)
        plsc.subcore_barrier()

        def body(j, carry):
            pltpu.sync_copy(ones_v, acc_sh.at[idx_v.at[j]], add=True)
            return carry

        lax.fori_loop(0, NCHUNK, body, 0)
        plsc.subcore_barrier()

        def out_copy(r0):
            pltpu.sync_copy(acc_sh.at[pl.ds(r0, ZR)], stage_v)
            pltpu.sync_copy(stage_v, out_hbm.at[c, pl.ds(r0, ZR)])

        _for_my_chunks(s, out_copy)

    return deg_kernel


def _make_agg_kernel(d: int):
    @functools.partial(
        pl.kernel,
        out_type=jax.ShapeDtypeStruct((2, N_NODES, d), jnp.float32),
        mesh=_sc_mesh(),
        scratch_types=[
            pltpu.VMEM((NCHUNK, CH), jnp.int32),
            pltpu.VMEM((NCHUNK, CH), jnp.int32),
            pltpu.VMEM((CH, d), jnp.float32),
            pltpu.VMEM((CH, d), jnp.float32),
            pltpu.VMEM((ZR, d), jnp.float32),
            pltpu.VMEM_SHARED((ACC_ROWS, d), jnp.float32),
            pltpu.SemaphoreType.DMA,
            pltpu.SemaphoreType.DMA,
            pltpu.SemaphoreType.DMA,
            pltpu.SemaphoreType.DMA,
        ],
        compiler_params=pltpu.CompilerParams(use_tc_tiling_on_sc=False),
    )
    def agg_kernel(y_hbm, src_hbm, dst_hbm, out_hbm,
                   src_v, dst_v, rows_a, rows_b, stage_v, acc_sh,
                   sga, sgb, ssa, ssb):
        c = lax.axis_index("c")
        s = lax.axis_index("s")
        wid = c * 16 + s

        _zero_fill(stage_v, ZR, d)
        _for_my_chunks(s, lambda r0: pltpu.sync_copy(
            stage_v, acc_sh.at[pl.ds(r0, ZR)]))

        pltpu.sync_copy(src_hbm.at[wid], src_v)
        pltpu.sync_copy(dst_hbm.at[wid], dst_v)
        plsc.subcore_barrier()

        def gather(j, buf, sem):
            pltpu.async_copy(y_hbm.at[src_v.at[j]], buf, sem)

        def scatter(j, buf, sem):
            pltpu.async_copy(buf, acc_sh.at[dst_v.at[j]], sem, add=True)

        def wait_g(buf, sem):
            pltpu.make_async_copy(y_hbm.at[src_v.at[0]], buf, sem).wait()

        def wait_s(buf, sem):
            pltpu.make_async_copy(buf, acc_sh.at[dst_v.at[0]], sem).wait()

        _pipeline(NCHUNK, gather, scatter, wait_g, wait_s,
                  rows_a, rows_b, sga, sgb, ssa, ssb)
        plsc.subcore_barrier()

        def out_copy(r0):
            pltpu.sync_copy(acc_sh.at[pl.ds(r0, ZR)], stage_v)
            pltpu.sync_copy(stage_v, out_hbm.at[c, pl.ds(r0, ZR)])

        _for_my_chunks(s, out_copy)

    return agg_kernel


NT2 = 16
NCHUNK2 = NW * NCHUNK // NT2


def _make_agg_dual_kernel():
    d = D2PAD

    @functools.partial(
        pl.kernel,
        out_type=jax.ShapeDtypeStruct((2, N_NODES, d), jnp.float32),
        mesh=_sc_mesh(),
        scratch_types=[
            pltpu.VMEM((NCHUNK2, CH), jnp.int32),
            pltpu.VMEM((NCHUNK2, CH), jnp.int32),
            pltpu.VMEM((CH, d), jnp.float32),
            pltpu.VMEM((CH, d), jnp.float32),
            pltpu.VMEM((ZR, d), jnp.float32),
            pltpu.VMEM_SHARED((ACC_ROWS, d), jnp.float32),
            pltpu.SemaphoreType.DMA,
            pltpu.SemaphoreType.DMA,
            pltpu.SemaphoreType.DMA,
            pltpu.SemaphoreType.DMA,
        ],
        compiler_params=pltpu.CompilerParams(use_tc_tiling_on_sc=False),
    )
    def agg_dual_kernel(ya_hbm, yb_hbm, src_hbm, dst_hbm, out_hbm,
                        src_v, dst_v, rows_a, rows_b, stage_v, acc_sh,
                        sga, sgb, ssa, ssb):
        c = lax.axis_index("c")
        s = lax.axis_index("s")

        _zero_fill(stage_v, ZR, d)
        _for_my_chunks(s, lambda r0: pltpu.sync_copy(
            stage_v, acc_sh.at[pl.ds(r0, ZR)]))

        pltpu.sync_copy(src_hbm.at[s], src_v)
        pltpu.sync_copy(dst_hbm.at[s], dst_v)
        plsc.subcore_barrier()

        def run(y_hbm):
            def gather(j, buf, sem):
                pltpu.async_copy(y_hbm.at[src_v.at[j]], buf, sem)

            def scatter(j, buf, sem):
                pltpu.async_copy(buf, acc_sh.at[dst_v.at[j]], sem, add=True)

            def wait_g(buf, sem):
                pltpu.make_async_copy(y_hbm.at[src_v.at[0]], buf, sem).wait()

            def wait_s(buf, sem):
                pltpu.make_async_copy(buf, acc_sh.at[dst_v.at[0]], sem).wait()

            _pipeline(NCHUNK2, gather, scatter, wait_g, wait_s,
                      rows_a, rows_b, sga, sgb, ssa, ssb)

        @pl.when(c == 0)
        def _():
            run(ya_hbm)

        @pl.when(c == 1)
        def _():
            run(yb_hbm)

        plsc.subcore_barrier()

        def out_copy(r0):
            pltpu.sync_copy(acc_sh.at[pl.ds(r0, ZR)], stage_v)
            pltpu.sync_copy(stage_v, out_hbm.at[c, pl.ds(r0, ZR)])

        _for_my_chunks(s, out_copy)

    return agg_dual_kernel


_deg_call = _make_deg_kernel()
_agg_dual_call = _make_agg_dual_kernel()
_agg48_call = _make_agg_kernel(DCLS)


_ROWS = 2000


def _dinv_block(degp_block):
    deg = degp_block[0, :, :1] + degp_block[1, :, :1] + 1.0
    return lax.rsqrt(deg)


def _tc1_body(x_ref, w1_ref, degp_ref, y1a_ref, y1b_ref):
    dinv = _dinv_block(degp_ref[...])
    xw = jnp.dot(x_ref[...], w1_ref[...], preferred_element_type=jnp.float32)
    y = xw * dinv
    y1a_ref[...] = y[:, :D2PAD]
    y1b_ref[...] = y[:, D2PAD:]


def _tc2_body(p1_ref, y1a_ref, y1b_ref, degp_ref, w2p_ref, b1_ref,
              y2p_ref):
    dinv = _dinv_block(degp_ref[...])
    p = p1_ref[...]
    agg = jnp.concatenate(
        [p[0] + y1a_ref[...], p[1] + y1b_ref[...]], axis=1)
    h = jnp.maximum(agg * dinv + b1_ref[...], 0.0)
    hw = jnp.dot(h, w2p_ref[...], preferred_element_type=jnp.float32)
    y2p_ref[...] = hw * dinv


def _tc3_body(p2_ref, y2p_ref, degp_ref, b2_ref, out_ref):
    dinv = _dinv_block(degp_ref[...])
    p = p2_ref[...]
    agg = (p[0] + p[1] + y2p_ref[...])[:, :NCLASS]
    logits = agg * dinv + b2_ref[...]
    m = jnp.max(logits, axis=1, keepdims=True)
    sh = logits - m
    lse = jnp.log(jnp.sum(jnp.exp(sh), axis=1, keepdims=True))
    out_ref[...] = sh - lse


def _row_spec(d):
    return pl.BlockSpec((_ROWS, d), lambda i: (i, 0))


def _part_spec(d):
    return pl.BlockSpec((2, _ROWS, d), lambda i: (0, i, 0))


def _full_spec(r, d):
    return pl.BlockSpec((r, d), lambda i: (0, 0))


_GRID = N_NODES // _ROWS

_tc1_call = pl.pallas_call(
    _tc1_body,
    grid=(_GRID,),
    in_specs=[_row_spec(D_IN), _full_spec(D_IN, NHID), _part_spec(16)],
    out_specs=[_row_spec(D2PAD), _row_spec(D2PAD)],
    out_shape=[jax.ShapeDtypeStruct((N_NODES, D2PAD), jnp.float32),
               jax.ShapeDtypeStruct((N_NODES, D2PAD), jnp.float32)],
)

_tc2_call = pl.pallas_call(
    _tc2_body,
    grid=(_GRID,),
    in_specs=[_part_spec(D2PAD), _row_spec(D2PAD),
              _row_spec(D2PAD), _part_spec(16),
              _full_spec(NHID, DCLS), _full_spec(1, NHID)],
    out_specs=_row_spec(DCLS),
    out_shape=jax.ShapeDtypeStruct((N_NODES, DCLS), jnp.float32),
)

_tc3_call = pl.pallas_call(
    _tc3_body,
    grid=(_GRID,),
    in_specs=[_part_spec(DCLS), _row_spec(DCLS), _part_spec(16),
              _full_spec(1, NCLASS)],
    out_specs=_row_spec(NCLASS),
    out_shape=jax.ShapeDtypeStruct((N_NODES, NCLASS), jnp.float32),
)


@jax.jit
def kernel(x, edge_index, W1, b1, W2, b2):
    pad_src = (jnp.arange(PADE, dtype=jnp.int32) * 97) % N_NODES
    pad_dst = N_NODES + (jnp.arange(PADE, dtype=jnp.int32) % ACC_PAD)
    src_p = jnp.concatenate([edge_index[0], pad_src])
    dst_p = jnp.concatenate([edge_index[1], pad_dst])
    src3 = src_p.reshape(NW, NCHUNK, CH)
    dst3 = dst_p.reshape(NW, NCHUNK, CH)
    srcT = src_p.reshape(NT2, NCHUNK2, CH)
    dstT = dst_p.reshape(NT2, NCHUNK2, CH)

    degp = _deg_call(dst3)
    y1a, y1b = _tc1_call(x, W1, degp)
    p1 = _agg_dual_call(y1a, y1b, srcT, dstT)
    w2p = jnp.pad(W2, ((0, 0), (0, DCLS - NCLASS)))
    y2p = _tc2_call(p1, y1a, y1b, degp, w2p, b1.reshape(1, NHID))
    p2 = _agg48_call(y2p, src3, dst3)
    return _tc3_call(p2, y2p, degp, b2.reshape(1, NCLASS))


# speedup vs baseline: 31.4591x; 1.2173x over previous
import functools

import jax
import jax.numpy as jnp
from jax import lax
from jax.experimental import pallas as pl
from jax.experimental.pallas import tpu as pltpu
import jax.experimental.pallas.tpu_sc as plsc

N_NODES = 10000
N_EDGES = 320000
D_IN = 128
NHID = 128
NCLASS = 40
D2PAD = 64
DCLS = 48

NW = 32
CH = 256
NCHUNK = -(-N_EDGES // (NW * CH))
EPW = NCHUNK * CH
PADE = NW * EPW - N_EDGES
ACC_PAD = 8
ACC_ROWS = N_NODES + ACC_PAD
ZR = 200
NZCH = N_NODES // ZR


def _sc_mesh():
    return plsc.VectorSubcoreMesh(core_axis_name="c", subcore_axis_name="s")


def _for_my_chunks(s, fn):
    def body(i, carry):
        k = s + 16 * i

        @pl.when(k < NZCH)
        def _():
            fn(k * ZR)

        return carry

    lax.fori_loop(0, (NZCH + 15) // 16, body, 0)


def _pipeline(nchunk, gather, scatter, wait_g, wait_s,
              rows_a, rows_b, sga, sgb, ssa, ssb):
    even = nchunk % 2 == 0
    pairs = (nchunk - 2) // 2 if even else (nchunk - 1) // 2
    gather(0, rows_a, sga)

    def body(i, carry):
        ja = 2 * i

        @pl.when(i > 0)
        def _():
            wait_s(rows_b, ssb)

        wait_g(rows_a, sga)
        gather(ja + 1, rows_b, sgb)
        scatter(ja, rows_a, ssa)
        wait_g(rows_b, sgb)
        wait_s(rows_a, ssa)
        gather(ja + 2, rows_a, sga)
        scatter(ja + 1, rows_b, ssb)
        return carry

    lax.fori_loop(0, pairs, body, 0)
    if pairs > 0:
        wait_s(rows_b, ssb)
    wait_g(rows_a, sga)
    if even:
        gather(nchunk - 1, rows_b, sgb)
        scatter(nchunk - 2, rows_a, ssa)
        wait_g(rows_b, sgb)
        wait_s(rows_a, ssa)
        scatter(nchunk - 1, rows_b, ssb)
        wait_s(rows_b, ssb)
    else:
        scatter(nchunk - 1, rows_a, ssa)
        wait_s(rows_a, ssa)


def _zero_fill(buf, rows, cols):
    nvec = rows * (cols // 16)

    def body(i, carry):
        r = i // (cols // 16)
        k = i % (cols // 16)
        buf[r, pl.ds(k * 16, 16)] = jnp.zeros((16,), jnp.float32)
        return carry

    lax.fori_loop(0, nvec, body, 0)


def _make_deg_kernel():
    @functools.partial(
        pl.kernel,
        out_type=jax.ShapeDtypeStruct((2, N_NODES, 16), jnp.float32),
        mesh=_sc_mesh(),
        scratch_types=[
            pltpu.VMEM((NCHUNK, CH), jnp.int32),
            pltpu.VMEM((CH, 16), jnp.float32),
            pltpu.VMEM((ZR, 16), jnp.float32),
            pltpu.VMEM_SHARED((ACC_ROWS, 16), jnp.float32),
        ],
        compiler_params=pltpu.CompilerParams(use_tc_tiling_on_sc=False),
    )
    def deg_kernel(dst_hbm, out_hbm, idx_v, ones_v, stage_v, acc_sh):
        c = lax.axis_index("c")
        s = lax.axis_index("s")
        wid = c * 16 + s

        _zero_fill(stage_v, ZR, 16)
        _for_my_chunks(s, lambda r0: pltpu.sync_copy(
            stage_v, acc_sh.at[pl.ds(r0, ZR)]))

        def ones_fill(i, carry):
            ones_v[i, :] = jnp.ones((16,), jnp.float32)
            return carry

        lax.fori_loop(0, CH, ones_fill, 0)

        pltpu.sync_copy(dst_hbm.at[wid], idx_v)
        plsc.subcore_barrier()

        def body(j, carry):
            pltpu.sync_copy(ones_v, acc_sh.at[idx_v.at[j]], add=True)
            return carry

        lax.fori_loop(0, NCHUNK, body, 0)
        plsc.subcore_barrier()

        def out_copy(r0):
            pltpu.sync_copy(acc_sh.at[pl.ds(r0, ZR)], stage_v)
            pltpu.sync_copy(stage_v, out_hbm.at[c, pl.ds(r0, ZR)])

        _for_my_chunks(s, out_copy)

    return deg_kernel


def _make_agg_kernel(d: int):
    @functools.partial(
        pl.kernel,
        out_type=jax.ShapeDtypeStruct((2, N_NODES, d), jnp.float32),
        mesh=_sc_mesh(),
        scratch_types=[
            pltpu.VMEM((NCHUNK, CH), jnp.int32),
            pltpu.VMEM((NCHUNK, CH), jnp.int32),
            pltpu.VMEM((CH, d), jnp.float32),
            pltpu.VMEM((CH, d), jnp.float32),
            pltpu.VMEM((ZR, d), jnp.float32),
            pltpu.VMEM_SHARED((ACC_ROWS, d), jnp.float32),
            pltpu.SemaphoreType.DMA,
            pltpu.SemaphoreType.DMA,
            pltpu.SemaphoreType.DMA,
            pltpu.SemaphoreType.DMA,
        ],
        compiler_params=pltpu.CompilerParams(use_tc_tiling_on_sc=False),
    )
    def agg_kernel(y_hbm, src_hbm, dst_hbm, out_hbm,
                   src_v, dst_v, rows_a, rows_b, stage_v, acc_sh,
                   sga, sgb, ssa, ssb):
        c = lax.axis_index("c")
        s = lax.axis_index("s")
        wid = c * 16 + s

        _zero_fill(stage_v, ZR, d)
        _for_my_chunks(s, lambda r0: pltpu.sync_copy(
            stage_v, acc_sh.at[pl.ds(r0, ZR)]))

        pltpu.sync_copy(src_hbm.at[wid], src_v)
        pltpu.sync_copy(dst_hbm.at[wid], dst_v)
        plsc.subcore_barrier()

        def gather(j, buf, sem):
            pltpu.async_copy(y_hbm.at[src_v.at[j]], buf, sem)

        def scatter(j, buf, sem):
            pltpu.async_copy(buf, acc_sh.at[dst_v.at[j]], sem, add=True)

        def wait_g(buf, sem):
            pltpu.make_async_copy(y_hbm.at[src_v.at[0]], buf, sem).wait()

        def wait_s(buf, sem):
            pltpu.make_async_copy(buf, acc_sh.at[dst_v.at[0]], sem).wait()

        _pipeline(NCHUNK, gather, scatter, wait_g, wait_s,
                  rows_a, rows_b, sga, sgb, ssa, ssb)
        plsc.subcore_barrier()

        def out_copy(r0):
            pltpu.sync_copy(acc_sh.at[pl.ds(r0, ZR)], stage_v)
            pltpu.sync_copy(stage_v, out_hbm.at[c, pl.ds(r0, ZR)])

        _for_my_chunks(s, out_copy)

    return agg_kernel


NT2 = 16
NCHUNK2 = NW * NCHUNK // NT2


def _make_agg_dual_kernel():
    d = D2PAD

    @functools.partial(
        pl.kernel,
        out_type=jax.ShapeDtypeStruct((2, N_NODES, d), jnp.float32),
        mesh=_sc_mesh(),
        scratch_types=[
            pltpu.VMEM((NCHUNK2, CH), jnp.int32),
            pltpu.VMEM((NCHUNK2, CH), jnp.int32),
            pltpu.VMEM((CH, d), jnp.float32),
            pltpu.VMEM((CH, d), jnp.float32),
            pltpu.VMEM((ZR, d), jnp.float32),
            pltpu.VMEM_SHARED((ACC_ROWS, d), jnp.float32),
            pltpu.SemaphoreType.DMA,
            pltpu.SemaphoreType.DMA,
            pltpu.SemaphoreType.DMA,
            pltpu.SemaphoreType.DMA,
        ],
        compiler_params=pltpu.CompilerParams(use_tc_tiling_on_sc=False),
    )
    def agg_dual_kernel(ya_hbm, yb_hbm, src_hbm, dst_hbm, out_hbm,
                        src_v, dst_v, rows_a, rows_b, stage_v, acc_sh,
                        sga, sgb, ssa, ssb):
        c = lax.axis_index("c")
        s = lax.axis_index("s")

        _zero_fill(stage_v, ZR, d)
        _for_my_chunks(s, lambda r0: pltpu.sync_copy(
            stage_v, acc_sh.at[pl.ds(r0, ZR)]))

        pltpu.sync_copy(src_hbm.at[s], src_v)
        pltpu.sync_copy(dst_hbm.at[s], dst_v)
        plsc.subcore_barrier()

        def run(y_hbm):
            def gather(j, buf, sem):
                pltpu.async_copy(y_hbm.at[src_v.at[j]], buf, sem)

            def scatter(j, buf, sem):
                pltpu.async_copy(buf, acc_sh.at[dst_v.at[j]], sem, add=True)

            def wait_g(buf, sem):
                pltpu.make_async_copy(y_hbm.at[src_v.at[0]], buf, sem).wait()

            def wait_s(buf, sem):
                pltpu.make_async_copy(buf, acc_sh.at[dst_v.at[0]], sem).wait()

            _pipeline(NCHUNK2, gather, scatter, wait_g, wait_s,
                      rows_a, rows_b, sga, sgb, ssa, ssb)

        @pl.when(c == 0)
        def _():
            run(ya_hbm)

        @pl.when(c == 1)
        def _():
            run(yb_hbm)

        plsc.subcore_barrier()

        def out_copy(r0):
            pltpu.sync_copy(acc_sh.at[pl.ds(r0, ZR)], stage_v)
            pltpu.sync_copy(stage_v, out_hbm.at[c, pl.ds(r0, ZR)])

        _for_my_chunks(s, out_copy)

    return agg_dual_kernel


_deg_call = _make_deg_kernel()
_agg_dual_call = _make_agg_dual_kernel()
_agg48_call = _make_agg_kernel(DCLS)


_ROWS = 2000


def _dinv_block(degp_block):
    deg = degp_block[0, :, :1] + degp_block[1, :, :1] + 1.0
    return lax.rsqrt(deg)


def _tc1_body(x_ref, w1_ref, degp_ref, y1a_ref, y1b_ref):
    dinv = _dinv_block(degp_ref[...])
    xw = jnp.dot(x_ref[...], w1_ref[...], preferred_element_type=jnp.float32)
    y = xw * dinv
    y1a_ref[...] = y[:, :D2PAD]
    y1b_ref[...] = y[:, D2PAD:]


def _tc2_body(p1_ref, y1a_ref, y1b_ref, degp_ref, w2p_ref, b1_ref,
              y2p_ref):
    dinv = _dinv_block(degp_ref[...])
    p = p1_ref[...]
    agg = jnp.concatenate(
        [p[0] + y1a_ref[...], p[1] + y1b_ref[...]], axis=1)
    h = jnp.maximum(agg * dinv + b1_ref[...], 0.0)
    hw = jnp.dot(h, w2p_ref[...], preferred_element_type=jnp.float32)
    y2p_ref[...] = hw * dinv


def _tc3_body(p2_ref, y2p_ref, degp_ref, b2_ref, out_ref):
    dinv = _dinv_block(degp_ref[...])
    p = p2_ref[...]
    agg = (p[0] + p[1] + y2p_ref[...])[:, :NCLASS]
    logits = agg * dinv + b2_ref[...]
    m = jnp.max(logits, axis=1, keepdims=True)
    sh = logits - m
    lse = jnp.log(jnp.sum(jnp.exp(sh), axis=1, keepdims=True))
    out_ref[...] = sh - lse


def _row_spec(d):
    return pl.BlockSpec((_ROWS, d), lambda i: (i, 0))


def _part_spec(d):
    return pl.BlockSpec((2, _ROWS, d), lambda i: (0, i, 0))


def _full_spec(r, d):
    return pl.BlockSpec((r, d), lambda i: (0, 0))


_GRID = N_NODES // _ROWS

_tc1_call = pl.pallas_call(
    _tc1_body,
    grid=(_GRID,),
    in_specs=[_row_spec(D_IN), _full_spec(D_IN, NHID), _part_spec(16)],
    out_specs=[_row_spec(D2PAD), _row_spec(D2PAD)],
    out_shape=[jax.ShapeDtypeStruct((N_NODES, D2PAD), jnp.float32),
               jax.ShapeDtypeStruct((N_NODES, D2PAD), jnp.float32)],
)

_tc2_call = pl.pallas_call(
    _tc2_body,
    grid=(_GRID,),
    in_specs=[_part_spec(D2PAD), _row_spec(D2PAD),
              _row_spec(D2PAD), _part_spec(16),
              _full_spec(NHID, DCLS), _full_spec(1, NHID)],
    out_specs=_row_spec(DCLS),
    out_shape=jax.ShapeDtypeStruct((N_NODES, DCLS), jnp.float32),
)

_tc3_call = pl.pallas_call(
    _tc3_body,
    grid=(_GRID,),
    in_specs=[_part_spec(DCLS), _row_spec(DCLS), _part_spec(16),
              _full_spec(1, NCLASS)],
    out_specs=_row_spec(NCLASS),
    out_shape=jax.ShapeDtypeStruct((N_NODES, NCLASS), jnp.float32),
)


@jax.jit
def kernel(x, edge_index, W1, b1, W2, b2):
    pad_src = (jnp.arange(PADE, dtype=jnp.int32) * 97) % N_NODES
    pad_dst = N_NODES + (jnp.arange(PADE, dtype=jnp.int32) % ACC_PAD)
    src_p = jnp.concatenate([edge_index[0], pad_src])
    dst_p = jnp.concatenate([edge_index[1], pad_dst])
    src3 = src_p.reshape(NW, NCHUNK, CH)
    dst3 = dst_p.reshape(NW, NCHUNK, CH)
    srcT = src_p.reshape(NT2, NCHUNK2, CH)
    dstT = dst_p.reshape(NT2, NCHUNK2, CH)

    degp = _deg_call(dst3)
    y1a, y1b = _tc1_call(x, W1, degp)
    p1 = _agg_dual_call(y1a, y1b, srcT, dstT)
    w2p = jnp.pad(W2, ((0, 0), (0, DCLS - NCLASS)))
    y2p = _tc2_call(p1, y1a, y1b, degp, w2p, b1.reshape(1, NHID))
    p2 = _agg48_call(y2p, src3, dst3)
    return _tc3_call(p2, y2p, degp, b2.reshape(1, NCLASS))


# speedup vs baseline: 32.5539x; 1.0348x over previous
import functools

import jax
import jax.numpy as jnp
from jax import lax
from jax.experimental import pallas as pl
from jax.experimental.pallas import tpu as pltpu
import jax.experimental.pallas.tpu_sc as plsc

N_NODES = 10000
N_EDGES = 320000
D_IN = 128
NHID = 128
NCLASS = 40
D2PAD = 64
DCLS = 48

NW = 32
CH = 256
CH2 = 512
E_TOT = 327680
NCHUNK = E_TOT // (NW * CH)
NCHUNK_L2 = E_TOT // (NW * CH2)
PADE = E_TOT - N_EDGES
ACC_PAD = 8
ACC_ROWS = N_NODES + ACC_PAD
ZR = 200
NZCH = N_NODES // ZR


def _sc_mesh():
    return plsc.VectorSubcoreMesh(core_axis_name="c", subcore_axis_name="s")


def _for_my_chunks(s, fn):
    def body(i, carry):
        k = s + 16 * i

        @pl.when(k < NZCH)
        def _():
            fn(k * ZR)

        return carry

    lax.fori_loop(0, (NZCH + 15) // 16, body, 0)


def _pipeline(nchunk, gather, scatter, wait_g, wait_s,
              rows_a, rows_b, sga, sgb, ssa, ssb):
    even = nchunk % 2 == 0
    pairs = (nchunk - 2) // 2 if even else (nchunk - 1) // 2
    gather(0, rows_a, sga)

    def body(i, carry):
        ja = 2 * i

        @pl.when(i > 0)
        def _():
            wait_s(rows_b, ssb)

        wait_g(rows_a, sga)
        gather(ja + 1, rows_b, sgb)
        scatter(ja, rows_a, ssa)
        wait_g(rows_b, sgb)
        wait_s(rows_a, ssa)
        gather(ja + 2, rows_a, sga)
        scatter(ja + 1, rows_b, ssb)
        return carry

    lax.fori_loop(0, pairs, body, 0)
    if pairs > 0:
        wait_s(rows_b, ssb)
    wait_g(rows_a, sga)
    if even:
        gather(nchunk - 1, rows_b, sgb)
        scatter(nchunk - 2, rows_a, ssa)
        wait_g(rows_b, sgb)
        wait_s(rows_a, ssa)
        scatter(nchunk - 1, rows_b, ssb)
        wait_s(rows_b, ssb)
    else:
        scatter(nchunk - 1, rows_a, ssa)
        wait_s(rows_a, ssa)


def _zero_fill(buf, rows, cols):
    nvec = rows * (cols // 16)

    def body(i, carry):
        r = i // (cols // 16)
        k = i % (cols // 16)
        buf[r, pl.ds(k * 16, 16)] = jnp.zeros((16,), jnp.float32)
        return carry

    lax.fori_loop(0, nvec, body, 0)


def _make_deg_kernel():
    @functools.partial(
        pl.kernel,
        out_type=jax.ShapeDtypeStruct((2, N_NODES, 16), jnp.float32),
        mesh=_sc_mesh(),
        scratch_types=[
            pltpu.VMEM((NCHUNK, CH), jnp.int32),
            pltpu.VMEM((CH, 16), jnp.float32),
            pltpu.VMEM((ZR, 16), jnp.float32),
            pltpu.VMEM_SHARED((ACC_ROWS, 16), jnp.float32),
        ],
        compiler_params=pltpu.CompilerParams(use_tc_tiling_on_sc=False),
    )
    def deg_kernel(dst_hbm, out_hbm, idx_v, ones_v, stage_v, acc_sh):
        c = lax.axis_index("c")
        s = lax.axis_index("s")
        wid = c * 16 + s

        _zero_fill(stage_v, ZR, 16)
        _for_my_chunks(s, lambda r0: pltpu.sync_copy(
            stage_v, acc_sh.at[pl.ds(r0, ZR)]))

        def ones_fill(i, carry):
            ones_v[i, :] = jnp.ones((16,), jnp.float32)
            return carry

        lax.fori_loop(0, CH, ones_fill, 0)

        pltpu.sync_copy(dst_hbm.at[wid], idx_v)
        plsc.subcore_barrier()

        def body(j, carry):
            pltpu.sync_copy(ones_v, acc_sh.at[idx_v.at[j]], add=True)
            return carry

        lax.fori_loop(0, NCHUNK, body, 0)
        plsc.subcore_barrier()

        def out_copy(r0):
            pltpu.sync_copy(acc_sh.at[pl.ds(r0, ZR)], stage_v)
            pltpu.sync_copy(stage_v, out_hbm.at[c, pl.ds(r0, ZR)])

        _for_my_chunks(s, out_copy)

    return deg_kernel


def _make_agg_kernel(d: int, ch: int, nchunk: int):
    @functools.partial(
        pl.kernel,
        out_type=jax.ShapeDtypeStruct((2, N_NODES, d), jnp.float32),
        mesh=_sc_mesh(),
        scratch_types=[
            pltpu.VMEM((nchunk, ch), jnp.int32),
            pltpu.VMEM((nchunk, ch), jnp.int32),
            pltpu.VMEM((ch, d), jnp.float32),
            pltpu.VMEM((ch, d), jnp.float32),
            pltpu.VMEM((ZR, d), jnp.float32),
            pltpu.VMEM_SHARED((ACC_ROWS, d), jnp.float32),
            pltpu.SemaphoreType.DMA,
            pltpu.SemaphoreType.DMA,
            pltpu.SemaphoreType.DMA,
            pltpu.SemaphoreType.DMA,
        ],
        compiler_params=pltpu.CompilerParams(use_tc_tiling_on_sc=False),
    )
    def agg_kernel(y_hbm, src_hbm, dst_hbm, out_hbm,
                   src_v, dst_v, rows_a, rows_b, stage_v, acc_sh,
                   sga, sgb, ssa, ssb):
        c = lax.axis_index("c")
        s = lax.axis_index("s")
        wid = c * 16 + s

        _zero_fill(stage_v, ZR, d)
        _for_my_chunks(s, lambda r0: pltpu.sync_copy(
            stage_v, acc_sh.at[pl.ds(r0, ZR)]))

        pltpu.sync_copy(src_hbm.at[wid], src_v)
        pltpu.sync_copy(dst_hbm.at[wid], dst_v)
        plsc.subcore_barrier()

        def gather(j, buf, sem):
            pltpu.async_copy(y_hbm.at[src_v.at[j]], buf, sem)

        def scatter(j, buf, sem):
            pltpu.async_copy(buf, acc_sh.at[dst_v.at[j]], sem, add=True)

        def wait_g(buf, sem):
            pltpu.make_async_copy(y_hbm.at[src_v.at[0]], buf, sem).wait()

        def wait_s(buf, sem):
            pltpu.make_async_copy(buf, acc_sh.at[dst_v.at[0]], sem).wait()

        _pipeline(nchunk, gather, scatter, wait_g, wait_s,
                  rows_a, rows_b, sga, sgb, ssa, ssb)
        plsc.subcore_barrier()

        def out_copy(r0):
            pltpu.sync_copy(acc_sh.at[pl.ds(r0, ZR)], stage_v)
            pltpu.sync_copy(stage_v, out_hbm.at[c, pl.ds(r0, ZR)])

        _for_my_chunks(s, out_copy)

    return agg_kernel


NT2 = 16
NCHUNK2 = NW * NCHUNK // NT2


def _make_agg_dual_kernel():
    d = D2PAD

    @functools.partial(
        pl.kernel,
        out_type=jax.ShapeDtypeStruct((2, N_NODES, d), jnp.float32),
        mesh=_sc_mesh(),
        scratch_types=[
            pltpu.VMEM((NCHUNK2, CH), jnp.int32),
            pltpu.VMEM((NCHUNK2, CH), jnp.int32),
            pltpu.VMEM((CH, d), jnp.float32),
            pltpu.VMEM((CH, d), jnp.float32),
            pltpu.VMEM((ZR, d), jnp.float32),
            pltpu.VMEM_SHARED((ACC_ROWS, d), jnp.float32),
            pltpu.SemaphoreType.DMA,
            pltpu.SemaphoreType.DMA,
            pltpu.SemaphoreType.DMA,
            pltpu.SemaphoreType.DMA,
        ],
        compiler_params=pltpu.CompilerParams(use_tc_tiling_on_sc=False),
    )
    def agg_dual_kernel(ya_hbm, yb_hbm, src_hbm, dst_hbm, out_hbm,
                        src_v, dst_v, rows_a, rows_b, stage_v, acc_sh,
                        sga, sgb, ssa, ssb):
        c = lax.axis_index("c")
        s = lax.axis_index("s")

        _zero_fill(stage_v, ZR, d)
        _for_my_chunks(s, lambda r0: pltpu.sync_copy(
            stage_v, acc_sh.at[pl.ds(r0, ZR)]))

        pltpu.sync_copy(src_hbm.at[s], src_v)
        pltpu.sync_copy(dst_hbm.at[s], dst_v)
        plsc.subcore_barrier()

        def run(y_hbm):
            def gather(j, buf, sem):
                pltpu.async_copy(y_hbm.at[src_v.at[j]], buf, sem)

            def scatter(j, buf, sem):
                pltpu.async_copy(buf, acc_sh.at[dst_v.at[j]], sem, add=True)

            def wait_g(buf, sem):
                pltpu.make_async_copy(y_hbm.at[src_v.at[0]], buf, sem).wait()

            def wait_s(buf, sem):
                pltpu.make_async_copy(buf, acc_sh.at[dst_v.at[0]], sem).wait()

            _pipeline(NCHUNK2, gather, scatter, wait_g, wait_s,
                      rows_a, rows_b, sga, sgb, ssa, ssb)

        @pl.when(c == 0)
        def _():
            run(ya_hbm)

        @pl.when(c == 1)
        def _():
            run(yb_hbm)

        plsc.subcore_barrier()

        def out_copy(r0):
            pltpu.sync_copy(acc_sh.at[pl.ds(r0, ZR)], stage_v)
            pltpu.sync_copy(stage_v, out_hbm.at[c, pl.ds(r0, ZR)])

        _for_my_chunks(s, out_copy)

    return agg_dual_kernel


_deg_call = _make_deg_kernel()
_agg_dual_call = _make_agg_dual_kernel()
_agg48_call = _make_agg_kernel(DCLS, CH2, NCHUNK_L2)


_ROWS = 2000


def _dinv_block(degp_block):
    deg = degp_block[0, :, :1] + degp_block[1, :, :1] + 1.0
    return lax.rsqrt(deg)


def _tc1_body(x_ref, w1_ref, degp_ref, y1a_ref, y1b_ref):
    dinv = _dinv_block(degp_ref[...])
    xw = jnp.dot(x_ref[...], w1_ref[...], preferred_element_type=jnp.float32)
    y = xw * dinv
    y1a_ref[...] = y[:, :D2PAD]
    y1b_ref[...] = y[:, D2PAD:]


def _tc2_body(p1_ref, y1a_ref, y1b_ref, degp_ref, w2p_ref, b1_ref,
              y2p_ref):
    dinv = _dinv_block(degp_ref[...])
    p = p1_ref[...]
    agg = jnp.concatenate(
        [p[0] + y1a_ref[...], p[1] + y1b_ref[...]], axis=1)
    h = jnp.maximum(agg * dinv + b1_ref[...], 0.0)
    hw = jnp.dot(h, w2p_ref[...], preferred_element_type=jnp.float32)
    y2p_ref[...] = hw * dinv


def _tc3_body(p2_ref, y2p_ref, degp_ref, b2_ref, out_ref):
    dinv = _dinv_block(degp_ref[...])
    p = p2_ref[...]
    agg = (p[0] + p[1] + y2p_ref[...])[:, :NCLASS]
    logits = agg * dinv + b2_ref[...]
    m = jnp.max(logits, axis=1, keepdims=True)
    sh = logits - m
    lse = jnp.log(jnp.sum(jnp.exp(sh), axis=1, keepdims=True))
    out_ref[...] = sh - lse


def _row_spec(d):
    return pl.BlockSpec((_ROWS, d), lambda i: (i, 0))


def _part_spec(d):
    return pl.BlockSpec((2, _ROWS, d), lambda i: (0, i, 0))


def _full_spec(r, d):
    return pl.BlockSpec((r, d), lambda i: (0, 0))


_GRID = N_NODES // _ROWS

_tc1_call = pl.pallas_call(
    _tc1_body,
    grid=(_GRID,),
    in_specs=[_row_spec(D_IN), _full_spec(D_IN, NHID), _part_spec(16)],
    out_specs=[_row_spec(D2PAD), _row_spec(D2PAD)],
    out_shape=[jax.ShapeDtypeStruct((N_NODES, D2PAD), jnp.float32),
               jax.ShapeDtypeStruct((N_NODES, D2PAD), jnp.float32)],
)

_tc2_call = pl.pallas_call(
    _tc2_body,
    grid=(_GRID,),
    in_specs=[_part_spec(D2PAD), _row_spec(D2PAD),
              _row_spec(D2PAD), _part_spec(16),
              _full_spec(NHID, DCLS), _full_spec(1, NHID)],
    out_specs=_row_spec(DCLS),
    out_shape=jax.ShapeDtypeStruct((N_NODES, DCLS), jnp.float32),
)

_tc3_call = pl.pallas_call(
    _tc3_body,
    grid=(_GRID,),
    in_specs=[_part_spec(DCLS), _row_spec(DCLS), _part_spec(16),
              _full_spec(1, NCLASS)],
    out_specs=_row_spec(NCLASS),
    out_shape=jax.ShapeDtypeStruct((N_NODES, NCLASS), jnp.float32),
)


@jax.jit
def kernel(x, edge_index, W1, b1, W2, b2):
    pad_src = (jnp.arange(PADE, dtype=jnp.int32) * 97) % N_NODES
    pad_dst = N_NODES + (jnp.arange(PADE, dtype=jnp.int32) % ACC_PAD)
    src_p = jnp.concatenate([edge_index[0], pad_src])
    dst_p = jnp.concatenate([edge_index[1], pad_dst])
    src3 = src_p.reshape(NW, NCHUNK, CH)
    dst3 = dst_p.reshape(NW, NCHUNK, CH)
    src3b = src_p.reshape(NW, NCHUNK_L2, CH2)
    dst3b = dst_p.reshape(NW, NCHUNK_L2, CH2)
    srcT = src_p.reshape(NT2, NCHUNK2, CH)
    dstT = dst_p.reshape(NT2, NCHUNK2, CH)

    degp = _deg_call(dst3)
    y1a, y1b = _tc1_call(x, W1, degp)
    p1 = _agg_dual_call(y1a, y1b, srcT, dstT)
    w2p = jnp.pad(W2, ((0, 0), (0, DCLS - NCLASS)))
    y2p = _tc2_call(p1, y1a, y1b, degp, w2p, b1.reshape(1, NHID))
    p2 = _agg48_call(y2p, src3b, dst3b)
    return _tc3_call(p2, y2p, degp, b2.reshape(1, NCLASS))


# speedup vs baseline: 33.3343x; 1.0240x over previous
import functools

import jax
import jax.numpy as jnp
from jax import lax
from jax.experimental import pallas as pl
from jax.experimental.pallas import tpu as pltpu
import jax.experimental.pallas.tpu_sc as plsc

N_NODES = 10000
N_EDGES = 320000
D_IN = 128
NHID = 128
NCLASS = 40
D2PAD = 64
DCLS = 48

NW = 32
CH = 256
CH2 = 512
E_TOT = 327680
NCHUNK = E_TOT // (NW * CH)
NCHUNK_L2 = E_TOT // (NW * CH2)
PADE = E_TOT - N_EDGES
ACC_PAD = 8
ACC_ROWS = N_NODES + ACC_PAD
ZR = 200
NZCH = N_NODES // ZR


def _sc_mesh():
    return plsc.VectorSubcoreMesh(core_axis_name="c", subcore_axis_name="s")


def _for_my_chunks(s, fn, zr=None):
    zr = ZR if zr is None else zr
    nzch = N_NODES // zr

    def body(i, carry):
        k = s + 16 * i

        @pl.when(k < nzch)
        def _():
            fn(k * zr)

        return carry

    lax.fori_loop(0, (nzch + 15) // 16, body, 0)


def _pipeline(nchunk, gather, scatter, wait_g, wait_s,
              rows_a, rows_b, sga, sgb, ssa, ssb):
    even = nchunk % 2 == 0
    pairs = (nchunk - 2) // 2 if even else (nchunk - 1) // 2
    gather(0, rows_a, sga)

    def body(i, carry):
        ja = 2 * i

        @pl.when(i > 0)
        def _():
            wait_s(rows_b, ssb)

        wait_g(rows_a, sga)
        gather(ja + 1, rows_b, sgb)
        scatter(ja, rows_a, ssa)
        wait_g(rows_b, sgb)
        wait_s(rows_a, ssa)
        gather(ja + 2, rows_a, sga)
        scatter(ja + 1, rows_b, ssb)
        return carry

    lax.fori_loop(0, pairs, body, 0)
    if pairs > 0:
        wait_s(rows_b, ssb)
    wait_g(rows_a, sga)
    if even:
        gather(nchunk - 1, rows_b, sgb)
        scatter(nchunk - 2, rows_a, ssa)
        wait_g(rows_b, sgb)
        wait_s(rows_a, ssa)
        scatter(nchunk - 1, rows_b, ssb)
        wait_s(rows_b, ssb)
    else:
        scatter(nchunk - 1, rows_a, ssa)
        wait_s(rows_a, ssa)


def _zero_fill(buf, rows, cols):
    nvec = rows * (cols // 16)

    def body(i, carry):
        r = i // (cols // 16)
        k = i % (cols // 16)
        buf[r, pl.ds(k * 16, 16)] = jnp.zeros((16,), jnp.float32)
        return carry

    lax.fori_loop(0, nvec, body, 0)


def _make_deg_kernel():
    @functools.partial(
        pl.kernel,
        out_type=jax.ShapeDtypeStruct((2, N_NODES, 16), jnp.float32),
        mesh=_sc_mesh(),
        scratch_types=[
            pltpu.VMEM((NCHUNK, CH), jnp.int32),
            pltpu.VMEM((CH, 16), jnp.float32),
            pltpu.VMEM((ZR, 16), jnp.float32),
            pltpu.VMEM_SHARED((ACC_ROWS, 16), jnp.float32),
        ],
        compiler_params=pltpu.CompilerParams(use_tc_tiling_on_sc=False),
    )
    def deg_kernel(dst_hbm, out_hbm, idx_v, ones_v, stage_v, acc_sh):
        c = lax.axis_index("c")
        s = lax.axis_index("s")
        wid = c * 16 + s

        _zero_fill(stage_v, ZR, 16)
        _for_my_chunks(s, lambda r0: pltpu.sync_copy(
            stage_v, acc_sh.at[pl.ds(r0, ZR)]))

        def ones_fill(i, carry):
            ones_v[i, :] = jnp.ones((16,), jnp.float32)
            return carry

        lax.fori_loop(0, CH, ones_fill, 0)

        pltpu.sync_copy(dst_hbm.at[wid], idx_v)
        plsc.subcore_barrier()

        def body(j, carry):
            pltpu.sync_copy(ones_v, acc_sh.at[idx_v.at[j]], add=True)
            return carry

        lax.fori_loop(0, NCHUNK, body, 0)
        plsc.subcore_barrier()

        def out_copy(r0):
            pltpu.sync_copy(acc_sh.at[pl.ds(r0, ZR)], stage_v)
            pltpu.sync_copy(stage_v, out_hbm.at[c, pl.ds(r0, ZR)])

        _for_my_chunks(s, out_copy)

    return deg_kernel


def _make_agg_kernel(d: int, ch: int, nchunk: int):
    @functools.partial(
        pl.kernel,
        out_type=jax.ShapeDtypeStruct((2, N_NODES, d), jnp.float32),
        mesh=_sc_mesh(),
        scratch_types=[
            pltpu.VMEM((nchunk, ch), jnp.int32),
            pltpu.VMEM((nchunk, ch), jnp.int32),
            pltpu.VMEM((ch, d), jnp.float32),
            pltpu.VMEM((ch, d), jnp.float32),
            pltpu.VMEM((ZR, d), jnp.float32),
            pltpu.VMEM_SHARED((ACC_ROWS, d), jnp.float32),
            pltpu.SemaphoreType.DMA,
            pltpu.SemaphoreType.DMA,
            pltpu.SemaphoreType.DMA,
            pltpu.SemaphoreType.DMA,
        ],
        compiler_params=pltpu.CompilerParams(use_tc_tiling_on_sc=False),
    )
    def agg_kernel(y_hbm, src_hbm, dst_hbm, out_hbm,
                   src_v, dst_v, rows_a, rows_b, stage_v, acc_sh,
                   sga, sgb, ssa, ssb):
        c = lax.axis_index("c")
        s = lax.axis_index("s")
        wid = c * 16 + s

        _zero_fill(stage_v, ZR, d)
        _for_my_chunks(s, lambda r0: pltpu.sync_copy(
            stage_v, acc_sh.at[pl.ds(r0, ZR)]))

        pltpu.sync_copy(src_hbm.at[wid], src_v)
        pltpu.sync_copy(dst_hbm.at[wid], dst_v)
        plsc.subcore_barrier()

        def gather(j, buf, sem):
            pltpu.async_copy(y_hbm.at[src_v.at[j]], buf, sem)

        def scatter(j, buf, sem):
            pltpu.async_copy(buf, acc_sh.at[dst_v.at[j]], sem, add=True)

        def wait_g(buf, sem):
            pltpu.make_async_copy(y_hbm.at[src_v.at[0]], buf, sem).wait()

        def wait_s(buf, sem):
            pltpu.make_async_copy(buf, acc_sh.at[dst_v.at[0]], sem).wait()

        _pipeline(nchunk, gather, scatter, wait_g, wait_s,
                  rows_a, rows_b, sga, sgb, ssa, ssb)
        plsc.subcore_barrier()

        def out_copy(r0):
            pltpu.sync_copy(acc_sh.at[pl.ds(r0, ZR)], stage_v)
            pltpu.sync_copy(stage_v, out_hbm.at[c, pl.ds(r0, ZR)])

        _for_my_chunks(s, out_copy)

    return agg_kernel


NT2 = 16
CH_D = 320
NCHUNK2 = E_TOT // (NT2 * CH_D)
ZR_D = 40


def _make_agg_dual_kernel():
    d = D2PAD

    @functools.partial(
        pl.kernel,
        out_type=jax.ShapeDtypeStruct((2, N_NODES, d), jnp.float32),
        mesh=_sc_mesh(),
        scratch_types=[
            pltpu.VMEM((NCHUNK2, CH_D), jnp.int32),
            pltpu.VMEM((NCHUNK2, CH_D), jnp.int32),
            pltpu.VMEM((CH_D, d), jnp.float32),
            pltpu.VMEM((CH_D, d), jnp.float32),
            pltpu.VMEM((ZR_D, d), jnp.float32),
            pltpu.VMEM_SHARED((ACC_ROWS, d), jnp.float32),
            pltpu.SemaphoreType.DMA,
            pltpu.SemaphoreType.DMA,
            pltpu.SemaphoreType.DMA,
            pltpu.SemaphoreType.DMA,
        ],
        compiler_params=pltpu.CompilerParams(use_tc_tiling_on_sc=False),
    )
    def agg_dual_kernel(ya_hbm, yb_hbm, src_hbm, dst_hbm, out_hbm,
                        src_v, dst_v, rows_a, rows_b, stage_v, acc_sh,
                        sga, sgb, ssa, ssb):
        c = lax.axis_index("c")
        s = lax.axis_index("s")

        _zero_fill(stage_v, ZR_D, d)
        _for_my_chunks(s, lambda r0: pltpu.sync_copy(
            stage_v, acc_sh.at[pl.ds(r0, ZR_D)]), zr=ZR_D)

        pltpu.sync_copy(src_hbm.at[s], src_v)
        pltpu.sync_copy(dst_hbm.at[s], dst_v)
        plsc.subcore_barrier()

        def run(y_hbm):
            def gather(j, buf, sem):
                pltpu.async_copy(y_hbm.at[src_v.at[j]], buf, sem)

            def scatter(j, buf, sem):
                pltpu.async_copy(buf, acc_sh.at[dst_v.at[j]], sem, add=True)

            def wait_g(buf, sem):
                pltpu.make_async_copy(y_hbm.at[src_v.at[0]], buf, sem).wait()

            def wait_s(buf, sem):
                pltpu.make_async_copy(buf, acc_sh.at[dst_v.at[0]], sem).wait()

            _pipeline(NCHUNK2, gather, scatter, wait_g, wait_s,
                      rows_a, rows_b, sga, sgb, ssa, ssb)

        @pl.when(c == 0)
        def _():
            run(ya_hbm)

        @pl.when(c == 1)
        def _():
            run(yb_hbm)

        plsc.subcore_barrier()

        def out_copy(r0):
            pltpu.sync_copy(acc_sh.at[pl.ds(r0, ZR_D)], stage_v)
            pltpu.sync_copy(stage_v, out_hbm.at[c, pl.ds(r0, ZR_D)])

        _for_my_chunks(s, out_copy, zr=ZR_D)

    return agg_dual_kernel


_deg_call = _make_deg_kernel()
_agg_dual_call = _make_agg_dual_kernel()
_agg48_call = _make_agg_kernel(DCLS, CH2, NCHUNK_L2)


_ROWS = 2000


def _dinv_block(degp_block):
    deg = degp_block[0, :, :1] + degp_block[1, :, :1] + 1.0
    return lax.rsqrt(deg)


def _tc1_body(x_ref, w1_ref, degp_ref, y1a_ref, y1b_ref):
    dinv = _dinv_block(degp_ref[...])
    xw = jnp.dot(x_ref[...], w1_ref[...], preferred_element_type=jnp.float32)
    y = xw * dinv
    y1a_ref[...] = y[:, :D2PAD]
    y1b_ref[...] = y[:, D2PAD:]


def _tc2_body(p1_ref, y1a_ref, y1b_ref, degp_ref, w2p_ref, b1_ref,
              y2p_ref):
    dinv = _dinv_block(degp_ref[...])
    p = p1_ref[...]
    agg = jnp.concatenate(
        [p[0] + y1a_ref[...], p[1] + y1b_ref[...]], axis=1)
    h = jnp.maximum(agg * dinv + b1_ref[...], 0.0)
    hw = jnp.dot(h, w2p_ref[...], preferred_element_type=jnp.float32)
    y2p_ref[...] = hw * dinv


def _tc3_body(p2_ref, y2p_ref, degp_ref, b2_ref, out_ref):
    dinv = _dinv_block(degp_ref[...])
    p = p2_ref[...]
    agg = (p[0] + p[1] + y2p_ref[...])[:, :NCLASS]
    logits = agg * dinv + b2_ref[...]
    m = jnp.max(logits, axis=1, keepdims=True)
    sh = logits - m
    lse = jnp.log(jnp.sum(jnp.exp(sh), axis=1, keepdims=True))
    out_ref[...] = sh - lse


def _row_spec(d):
    return pl.BlockSpec((_ROWS, d), lambda i: (i, 0))


def _part_spec(d):
    return pl.BlockSpec((2, _ROWS, d), lambda i: (0, i, 0))


def _full_spec(r, d):
    return pl.BlockSpec((r, d), lambda i: (0, 0))


_GRID = N_NODES // _ROWS

_tc1_call = pl.pallas_call(
    _tc1_body,
    grid=(_GRID,),
    in_specs=[_row_spec(D_IN), _full_spec(D_IN, NHID), _part_spec(16)],
    out_specs=[_row_spec(D2PAD), _row_spec(D2PAD)],
    out_shape=[jax.ShapeDtypeStruct((N_NODES, D2PAD), jnp.float32),
               jax.ShapeDtypeStruct((N_NODES, D2PAD), jnp.float32)],
)

_tc2_call = pl.pallas_call(
    _tc2_body,
    grid=(_GRID,),
    in_specs=[_part_spec(D2PAD), _row_spec(D2PAD),
              _row_spec(D2PAD), _part_spec(16),
              _full_spec(NHID, DCLS), _full_spec(1, NHID)],
    out_specs=_row_spec(DCLS),
    out_shape=jax.ShapeDtypeStruct((N_NODES, DCLS), jnp.float32),
)

_tc3_call = pl.pallas_call(
    _tc3_body,
    grid=(_GRID,),
    in_specs=[_part_spec(DCLS), _row_spec(DCLS), _part_spec(16),
              _full_spec(1, NCLASS)],
    out_specs=_row_spec(NCLASS),
    out_shape=jax.ShapeDtypeStruct((N_NODES, NCLASS), jnp.float32),
)


@jax.jit
def kernel(x, edge_index, W1, b1, W2, b2):
    pad_src = (jnp.arange(PADE, dtype=jnp.int32) * 97) % N_NODES
    pad_dst = N_NODES + (jnp.arange(PADE, dtype=jnp.int32) % ACC_PAD)
    src_p = jnp.concatenate([edge_index[0], pad_src])
    dst_p = jnp.concatenate([edge_index[1], pad_dst])
    src3 = src_p.reshape(NW, NCHUNK, CH)
    dst3 = dst_p.reshape(NW, NCHUNK, CH)
    src3b = src_p.reshape(NW, NCHUNK_L2, CH2)
    dst3b = dst_p.reshape(NW, NCHUNK_L2, CH2)
    srcT = src_p.reshape(NT2, NCHUNK2, CH_D)
    dstT = dst_p.reshape(NT2, NCHUNK2, CH_D)

    degp = _deg_call(dst3)
    y1a, y1b = _tc1_call(x, W1, degp)
    p1 = _agg_dual_call(y1a, y1b, srcT, dstT)
    w2p = jnp.pad(W2, ((0, 0), (0, DCLS - NCLASS)))
    y2p = _tc2_call(p1, y1a, y1b, degp, w2p, b1.reshape(1, NHID))
    p2 = _agg48_call(y2p, src3b, dst3b)
    return _tc3_call(p2, y2p, degp, b2.reshape(1, NCLASS))
